# Initial kernel scaffold; baseline (speedup 1.0000x reference)
import jax
import jax.numpy as jnp
from jax.experimental import pallas as pl


def kernel(nodes, edges, senders, receivers, W1, b1, W2, b2, W3, b3, W4, b4, W5, b5):
    raise NotImplementedError("write your pallas kernel here")


# speedup vs baseline: 5.4182x; 5.4182x over previous
import functools

import jax
import jax.numpy as jnp
from jax import lax
from jax.experimental import pallas as pl
from jax.experimental.pallas import tpu as pltpu
from jax.experimental.pallas import tpu_sc as plsc

N = 10000
E = 320000
D = 128
DE = 16

NC = 2
NS = 16
NW = NC * NS
EPW = E // NW
BE = 80
NBLK = EPW // BE
NPAD = 10240
ROWS_PT = NPAD // NS

@functools.cache
def _mesh():
    return plsc.VectorSubcoreMesh(
        core_axis_name="c", subcore_axis_name="s",
        num_cores=NC, num_subcores=NS)


def _node_mm_body(x_ref, w1, b1, w2, b2, w5, b5, w4,
                  a_out, b_out, m_out, as_out, bs_out):
    x = x_ref[...]
    a = jnp.dot(x, w1[...], preferred_element_type=jnp.float32) + b1[...]
    b = jnp.dot(x, w2[...], preferred_element_type=jnp.float32) + b2[...]
    m = jnp.dot(x, w5[...], preferred_element_type=jnp.float32) + b5[...]
    a_out[...] = a
    b_out[...] = b
    m_out[...] = m
    w4v = w4[...]
    as_out[...] = jnp.dot(a, w4v, preferred_element_type=jnp.float32)
    bs_out[...] = jnp.dot(b, w4v, preferred_element_type=jnp.float32)


def _edge_mm_body(e_ref, w3, b3, w4, b4, ef0_out, es_out):
    e = e_ref[...]
    ef0 = jnp.dot(e, w3[...], preferred_element_type=jnp.float32) + b3[...]
    ef0_out[...] = ef0
    es_out[...] = jnp.dot(ef0, w4[...], preferred_element_type=jnp.float32) + b4[...]


def _sum_cores_body(acc_ref, out_ref):
    out_ref[...] = acc_ref[0] + acc_ref[1]


def _node_mm(nodes, W1, b1, W2, b2, W5, b5, W4):
    blk = 400
    grid = N // blk
    full = lambda shape: pl.BlockSpec(shape, lambda i: (0, 0))
    return pl.pallas_call(
        _node_mm_body,
        grid=(grid,),
        in_specs=[
            pl.BlockSpec((blk, D), lambda i: (i, 0)),
            full((D, D)), full((1, D)),
            full((D, D)), full((1, D)),
            full((D, D)), full((1, D)),
            full((D, 1)),
        ],
        out_specs=[
            pl.BlockSpec((blk, D), lambda i: (i, 0)),
            pl.BlockSpec((blk, D), lambda i: (i, 0)),
            pl.BlockSpec((blk, D), lambda i: (i, 0)),
            pl.BlockSpec((blk, 1), lambda i: (i, 0)),
            pl.BlockSpec((blk, 1), lambda i: (i, 0)),
        ],
        out_shape=[
            jax.ShapeDtypeStruct((N, D), jnp.float32),
            jax.ShapeDtypeStruct((N, D), jnp.float32),
            jax.ShapeDtypeStruct((N, D), jnp.float32),
            jax.ShapeDtypeStruct((N, 1), jnp.float32),
            jax.ShapeDtypeStruct((N, 1), jnp.float32),
        ],
    )(nodes, W1, b1.reshape(1, D), W2, b2.reshape(1, D),
      W5, b5.reshape(1, D), W4)


def _edge_mm(edges, W3, b3, W4, b4):
    blk = 2000
    grid = E // blk
    full = lambda shape: pl.BlockSpec(shape, lambda i: (0, 0))
    return pl.pallas_call(
        _edge_mm_body,
        grid=(grid,),
        in_specs=[
            pl.BlockSpec((blk, DE), lambda i: (i, 0)),
            full((DE, D)), full((1, D)),
            full((D, 1)), full((1, 1)),
        ],
        out_specs=[
            pl.BlockSpec((blk, D), lambda i: (i, 0)),
            pl.BlockSpec((blk, 1), lambda i: (i, 0)),
        ],
        out_shape=[
            jax.ShapeDtypeStruct((E, D), jnp.float32),
            jax.ShapeDtypeStruct((E, 1), jnp.float32),
        ],
    )(edges, W3, b3.reshape(1, D), W4, b4.reshape(1, 1))


def _sum_cores(acc):
    blk = 2048
    grid = NPAD // blk
    return pl.pallas_call(
        _sum_cores_body,
        grid=(grid,),
        in_specs=[pl.BlockSpec((NC, blk, D), lambda i: (0, i, 0))],
        out_specs=pl.BlockSpec((blk, D), lambda i: (i, 0)),
        out_shape=jax.ShapeDtypeStruct((NPAD, D), jnp.float32),
    )(acc)


def _edge_phase_body(senders, receivers, a_t, b_t, ef0, a_sc, b_sc, e_sc,
                     ef_out, p_out, den_out,
                     sidx, ridx, rows_a, rows_b, ef0b, escb, pbuf,
                     atab, btab, dtile, dsp, sem1, sem2, sem3):
    cid = lax.axis_index("c")
    sid = lax.axis_index("s")
    wid = sid * NC + cid

    def _z(i, _):
        dtile[pl.ds(i * 16, 16)] = jnp.zeros((16,), jnp.float32)
        return 0
    lax.fori_loop(0, ROWS_PT // 16, _z, 0)
    pltpu.sync_copy(dtile, dsp.at[pl.ds(sid * ROWS_PT, ROWS_PT)])

    pltpu.sync_copy(a_sc, atab.at[pl.ds(0, N)])
    pltpu.sync_copy(b_sc, btab.at[pl.ds(0, N)])
    plsc.subcore_barrier()

    base0 = wid * EPW

    def _blk(bi, _):
        base = base0 + bi * BE
        pltpu.sync_copy(senders.at[pl.ds(base, BE)], sidx)
        pltpu.sync_copy(receivers.at[pl.ds(base, BE)], ridx)
        pltpu.sync_copy(e_sc.at[pl.ds(base, BE)], escb)
        cp_a = pltpu.async_copy(a_t.at[sidx], rows_a, sem1)
        cp_b = pltpu.async_copy(b_t.at[ridx], rows_b, sem2)
        cp_e = pltpu.async_copy(ef0.at[pl.ds(base, BE)], ef0b, sem3)

        def _g(gi, _):
            sl = pl.ds(gi * 16, 16)
            av = plsc.load_gather(atab, [sidx[sl]])
            bv = plsc.load_gather(btab, [ridx[sl]])
            logit = av + bv + escb[sl]
            logit = jnp.where(logit > 0, logit, logit * jnp.float32(0.01))
            pbuf[sl] = jnp.exp(logit)
            return 0
        lax.fori_loop(0, BE // 16, _g, 0)

        cp_a.wait()
        cp_b.wait()
        cp_e.wait()

        def _row(i, _):
            for j in range(D // 16):
                sl = pl.ds(j * 16, 16)
                ef0b[i, sl] = ef0b[i, sl] + rows_a[i, sl] + rows_b[i, sl]
            return 0
        lax.fori_loop(0, BE, _row, 0)

        pltpu.sync_copy(ef0b, ef_out.at[pl.ds(base, BE)])
        pltpu.sync_copy(pbuf, p_out.at[pl.ds(base, BE)])
        pltpu.sync_copy(pbuf, dsp.at[ridx], add=True)
        return 0

    lax.fori_loop(0, NBLK, _blk, 0)

    plsc.subcore_barrier()
    pltpu.sync_copy(dsp.at[pl.ds(sid * ROWS_PT, ROWS_PT)], dtile)
    pltpu.sync_copy(dtile, den_out.at[cid, pl.ds(sid * ROWS_PT, ROWS_PT)])


@functools.cache
def _edge_phase():
    return pl.kernel(
        _edge_phase_body,
        out_type=(
            jax.ShapeDtypeStruct((E, D), jnp.float32),
            jax.ShapeDtypeStruct((E,), jnp.float32),
            jax.ShapeDtypeStruct((NC, NPAD), jnp.float32),
        ),
        mesh=_mesh(),
        scratch_types=[
        pltpu.VMEM((BE,), jnp.int32),
        pltpu.VMEM((BE,), jnp.int32),
        pltpu.VMEM((BE, D), jnp.float32),
        pltpu.VMEM((BE, D), jnp.float32),
        pltpu.VMEM((BE, D), jnp.float32),
        pltpu.VMEM((BE,), jnp.float32),
        pltpu.VMEM((BE,), jnp.float32),
        pltpu.VMEM((NPAD,), jnp.float32),
        pltpu.VMEM((NPAD,), jnp.float32),
        pltpu.VMEM((ROWS_PT,), jnp.float32),
        pltpu.VMEM_SHARED((NPAD,), jnp.float32),
            pltpu.SemaphoreType.DMA,
            pltpu.SemaphoreType.DMA,
            pltpu.SemaphoreType.DMA,
        ],
        compiler_params=pltpu.CompilerParams(needs_layout_passes=False),
    )


def _msg_phase_body(senders, receivers, m_t, p_in, den,
                    acc_out,
                    sidx, ridx, rows_m, pbuf, wbuf, dtab, dtmp, acc_sp, sem1):
    cid = lax.axis_index("c")
    sid = lax.axis_index("s")
    wid = sid * NC + cid

    pltpu.sync_copy(den.at[0], dtab)
    pltpu.sync_copy(den.at[1], dtmp)

    def _d(i, _):
        sl = pl.ds(i * 16, 16)
        dtab[sl] = dtab[sl] + dtmp[sl]
        return 0
    lax.fori_loop(0, NPAD // 16, _d, 0)

    def _zrow(i, _):
        for j in range(D // 16):
            rows_m[i, pl.ds(j * 16, 16)] = jnp.zeros((16,), jnp.float32)
        return 0
    lax.fori_loop(0, BE, _zrow, 0)
    for k in range(ROWS_PT // BE):
        pltpu.sync_copy(rows_m, acc_sp.at[pl.ds(sid * ROWS_PT + k * BE, BE)])
    plsc.subcore_barrier()

    base0 = wid * EPW

    def _blk(bi, _):
        base = base0 + bi * BE
        pltpu.sync_copy(senders.at[pl.ds(base, BE)], sidx)
        pltpu.sync_copy(receivers.at[pl.ds(base, BE)], ridx)
        pltpu.sync_copy(p_in.at[pl.ds(base, BE)], pbuf)
        cp_m = pltpu.async_copy(m_t.at[sidx], rows_m, sem1)

        def _g(gi, _):
            sl = pl.ds(gi * 16, 16)
            dv = plsc.load_gather(dtab, [ridx[sl]])
            wbuf[sl] = pbuf[sl] / dv
            return 0
        lax.fori_loop(0, BE // 16, _g, 0)

        cp_m.wait()

        def _row(i, _):
            wv = plsc.load_gather(wbuf, [jnp.full((16,), i, jnp.int32)])
            for j in range(D // 16):
                sl = pl.ds(j * 16, 16)
                rows_m[i, sl] = rows_m[i, sl] * wv
            return 0
        lax.fori_loop(0, BE, _row, 0)

        pltpu.sync_copy(rows_m, acc_sp.at[ridx], add=True)
        return 0

    lax.fori_loop(0, NBLK, _blk, 0)

    plsc.subcore_barrier()
    for k in range(ROWS_PT // BE):
        sl = pl.ds(sid * ROWS_PT + k * BE, BE)
        pltpu.sync_copy(acc_sp.at[sl], rows_m)
        pltpu.sync_copy(rows_m, acc_out.at[cid, sl])


@functools.cache
def _msg_phase():
    return pl.kernel(
        _msg_phase_body,
        out_type=jax.ShapeDtypeStruct((NC, NPAD, D), jnp.float32),
        mesh=_mesh(),
        scratch_types=[
        pltpu.VMEM((BE,), jnp.int32),
        pltpu.VMEM((BE,), jnp.int32),
        pltpu.VMEM((BE, D), jnp.float32),
        pltpu.VMEM((BE,), jnp.float32),
        pltpu.VMEM((BE,), jnp.float32),
        pltpu.VMEM((NPAD,), jnp.float32),
        pltpu.VMEM((NPAD,), jnp.float32),
        pltpu.VMEM_SHARED((NPAD, D), jnp.float32),
        pltpu.SemaphoreType.DMA,
        ],
        compiler_params=pltpu.CompilerParams(needs_layout_passes=False),
    )


def kernel(nodes, edges, senders, receivers,
           W1, b1, W2, b2, W3, b3, W4, b4, W5, b5):
    s32 = senders.astype(jnp.int32)
    r32 = receivers.astype(jnp.int32)

    a_t, b_t, m_t, a_sc, b_sc = _node_mm(nodes, W1, b1, W2, b2, W5, b5, W4)
    ef0, e_sc = _edge_mm(edges, W3, b3, W4, b4)

    ef, p, den = _edge_phase()(
        s32, r32, a_t, b_t, ef0,
        a_sc.reshape(N), b_sc.reshape(N), e_sc.reshape(E))
    acc = _msg_phase()(s32, r32, m_t, p, den)
    new_nodes = _sum_cores(acc)[:N]
    return new_nodes, ef


# speedup vs baseline: 8.7657x; 1.6178x over previous
import functools
from types import SimpleNamespace

import jax
import jax.numpy as jnp
from jax import lax
from jax.experimental import pallas as pl
from jax.experimental.pallas import tpu as pltpu
from jax.experimental.pallas import tpu_sc as plsc

N = 10000
E = 320000
D = 128
DE = 16

NC = 2
NS = 16
NW = NC * NS
EPW = E // NW
BE = 80
NBLK = EPW // BE
NPAD = 10240
ROWS_PT = NPAD // NS


@functools.cache
def _mesh():
    return plsc.VectorSubcoreMesh(
        core_axis_name="c", subcore_axis_name="s",
        num_cores=NC, num_subcores=NS)


def _node_mm_body(x_ref, w1, b1, w2, b2, w5, b5, w4,
                  a_out, b_out, m_out, as_out, bs_out):
    x = x_ref[...]
    a = jnp.dot(x, w1[...], preferred_element_type=jnp.float32) + b1[...]
    b = jnp.dot(x, w2[...], preferred_element_type=jnp.float32) + b2[...]
    m = jnp.dot(x, w5[...], preferred_element_type=jnp.float32) + b5[...]
    a_out[...] = a
    b_out[...] = b
    m_out[...] = m
    w4v = w4[...]
    as_out[...] = jnp.dot(a, w4v, preferred_element_type=jnp.float32)
    bs_out[...] = jnp.dot(b, w4v, preferred_element_type=jnp.float32)


def _edge_mm_body(e_ref, w3, b3, w4, b4, ef0_out, es_out):
    e = e_ref[...]
    ef0 = jnp.dot(e, w3[...], preferred_element_type=jnp.float32) + b3[...]
    ef0_out[...] = ef0
    es_out[...] = jnp.dot(ef0, w4[...], preferred_element_type=jnp.float32) + b4[...]


def _sum_cores_body(acc_ref, out_ref):
    out_ref[...] = acc_ref[0] + acc_ref[1]


def _node_mm(nodes, W1, b1, W2, b2, W5, b5, W4):
    blk = 400
    grid = N // blk
    full = lambda shape: pl.BlockSpec(shape, lambda i: (0, 0))
    return pl.pallas_call(
        _node_mm_body,
        grid=(grid,),
        in_specs=[
            pl.BlockSpec((blk, D), lambda i: (i, 0)),
            full((D, D)), full((1, D)),
            full((D, D)), full((1, D)),
            full((D, D)), full((1, D)),
            full((D, 1)),
        ],
        out_specs=[
            pl.BlockSpec((blk, D), lambda i: (i, 0)),
            pl.BlockSpec((blk, D), lambda i: (i, 0)),
            pl.BlockSpec((blk, D), lambda i: (i, 0)),
            pl.BlockSpec((blk, 1), lambda i: (i, 0)),
            pl.BlockSpec((blk, 1), lambda i: (i, 0)),
        ],
        out_shape=[
            jax.ShapeDtypeStruct((N, D), jnp.float32),
            jax.ShapeDtypeStruct((N, D), jnp.float32),
            jax.ShapeDtypeStruct((N, D), jnp.float32),
            jax.ShapeDtypeStruct((N, 1), jnp.float32),
            jax.ShapeDtypeStruct((N, 1), jnp.float32),
        ],
    )(nodes, W1, b1.reshape(1, D), W2, b2.reshape(1, D),
      W5, b5.reshape(1, D), W4)


def _edge_mm(edges, W3, b3, W4, b4):
    blk = 2000
    grid = E // blk
    full = lambda shape: pl.BlockSpec(shape, lambda i: (0, 0))
    return pl.pallas_call(
        _edge_mm_body,
        grid=(grid,),
        in_specs=[
            pl.BlockSpec((blk, DE), lambda i: (i, 0)),
            full((DE, D)), full((1, D)),
            full((D, 1)), full((1, 1)),
        ],
        out_specs=[
            pl.BlockSpec((blk, D), lambda i: (i, 0)),
            pl.BlockSpec((blk, 1), lambda i: (i, 0)),
        ],
        out_shape=[
            jax.ShapeDtypeStruct((E, D), jnp.float32),
            jax.ShapeDtypeStruct((E, 1), jnp.float32),
        ],
    )(edges, W3, b3.reshape(1, D), W4, b4.reshape(1, 1))


def _sum_cores(acc):
    blk = 2048
    grid = NPAD // blk
    return pl.pallas_call(
        _sum_cores_body,
        grid=(grid,),
        in_specs=[pl.BlockSpec((NC, blk, D), lambda i: (0, i, 0))],
        out_specs=pl.BlockSpec((blk, D), lambda i: (i, 0)),
        out_shape=jax.ShapeDtypeStruct((NPAD, D), jnp.float32),
    )(acc)


def _edge_phase_body(senders, receivers, a_t, b_t, ef0, a_sc, b_sc, e_sc,
                     ef_out, p_out, den_out, *scr):
    s0 = SimpleNamespace(sidx=scr[0], ridx=scr[1], escb=scr[2], pbuf=scr[3],
                         rows_a=scr[4], rows_b=scr[5], ef0b=scr[6],
                         semi=scr[18], semg=scr[19], semw=scr[20])
    s1 = SimpleNamespace(sidx=scr[7], ridx=scr[8], escb=scr[9], pbuf=scr[10],
                         rows_a=scr[11], rows_b=scr[12], ef0b=scr[13],
                         semi=scr[21], semg=scr[22], semw=scr[23])
    atab, btab, dtile, dsp = scr[14], scr[15], scr[16], scr[17]

    cid = lax.axis_index("c")
    sid = lax.axis_index("s")
    wid = sid * NC + cid
    base0 = wid * EPW

    def _z(i, _):
        dtile[pl.ds(i * 16, 16)] = jnp.zeros((16,), jnp.float32)
        return 0
    lax.fori_loop(0, ROWS_PT // 16, _z, 0)
    pltpu.sync_copy(dtile, dsp.at[pl.ds(sid * ROWS_PT, ROWS_PT)])

    pltpu.sync_copy(a_sc, atab.at[pl.ds(0, N)])
    pltpu.sync_copy(b_sc, btab.at[pl.ds(0, N)])
    plsc.subcore_barrier()

    def idx_copies(j, S):
        base = base0 + j * BE
        return (
            pltpu.make_async_copy(senders.at[pl.ds(base, BE)], S.sidx, S.semi),
            pltpu.make_async_copy(receivers.at[pl.ds(base, BE)], S.ridx, S.semi),
            pltpu.make_async_copy(e_sc.at[pl.ds(base, BE)], S.escb, S.semi),
        )

    def gather_copies(j, S):
        base = base0 + j * BE
        return (
            pltpu.make_async_copy(a_t.at[S.sidx], S.rows_a, S.semg),
            pltpu.make_async_copy(b_t.at[S.ridx], S.rows_b, S.semg),
            pltpu.make_async_copy(ef0.at[pl.ds(base, BE)], S.ef0b, S.semg),
        )

    def ef_write(j, S):
        base = base0 + j * BE
        return pltpu.make_async_copy(S.ef0b, ef_out.at[pl.ds(base, BE)], S.semw)

    def halfstep(j, sc, sn, next1, next2, drainw):
        if next1:
            for c in idx_copies(j + 1, sn):
                c.wait()
            if drainw:
                ef_write(j - 1, sn).wait()
            for c in gather_copies(j + 1, sn):
                c.start()
        elif drainw:
            ef_write(j - 1, sn).wait()

        def _g(gi, _):
            sl = pl.ds(gi * 16, 16)
            av = plsc.load_gather(atab, [sc.sidx[sl]])
            bv = plsc.load_gather(btab, [sc.ridx[sl]])
            logit = av + bv + sc.escb[sl]
            logit = jnp.where(logit > 0, logit, logit * jnp.float32(0.01))
            sc.pbuf[sl] = jnp.exp(logit)
            return 0
        lax.fori_loop(0, BE // 16, _g, 0)

        for c in gather_copies(j, sc):
            c.wait()

        def _row(i, _):
            for k in range(D // 16):
                sl = pl.ds(k * 16, 16)
                sc.ef0b[i, sl] = sc.ef0b[i, sl] + sc.rows_a[i, sl] + sc.rows_b[i, sl]
            return 0
        lax.fori_loop(0, BE, _row, 0)

        ef_write(j, sc).start()
        base = base0 + j * BE
        pltpu.sync_copy(sc.pbuf, p_out.at[pl.ds(base, BE)])
        pltpu.sync_copy(sc.pbuf, dsp.at[sc.ridx], add=True)
        if next2:
            for c in idx_copies(j + 2, sc):
                c.start()

    for c in idx_copies(0, s0):
        c.start()
    for c in idx_copies(0, s0):
        c.wait()
    for c in gather_copies(0, s0):
        c.start()
    for c in idx_copies(1, s1):
        c.start()

    halfstep(0, s0, s1, True, True, False)

    def _pair(k, _):
        halfstep(2 * k + 1, s1, s0, True, True, True)
        halfstep(2 * k + 2, s0, s1, True, True, True)
        return 0
    lax.fori_loop(0, (NBLK - 3) // 2, _pair, 0)

    halfstep(NBLK - 2, s1, s0, True, False, True)
    halfstep(NBLK - 1, s0, s1, False, False, True)
    ef_write(NBLK - 1, s0).wait()

    plsc.subcore_barrier()
    pltpu.sync_copy(dsp.at[pl.ds(sid * ROWS_PT, ROWS_PT)], dtile)
    pltpu.sync_copy(dtile, den_out.at[cid, pl.ds(sid * ROWS_PT, ROWS_PT)])


@functools.cache
def _edge_phase():
    sets = []
    for _ in range(2):
        sets += [
            pltpu.VMEM((BE,), jnp.int32),
            pltpu.VMEM((BE,), jnp.int32),
            pltpu.VMEM((BE,), jnp.float32),
            pltpu.VMEM((BE,), jnp.float32),
            pltpu.VMEM((BE, D), jnp.float32),
            pltpu.VMEM((BE, D), jnp.float32),
            pltpu.VMEM((BE, D), jnp.float32),
        ]
    return pl.kernel(
        _edge_phase_body,
        out_type=(
            jax.ShapeDtypeStruct((E, D), jnp.float32),
            jax.ShapeDtypeStruct((E,), jnp.float32),
            jax.ShapeDtypeStruct((NC, NPAD), jnp.float32),
        ),
        mesh=_mesh(),
        scratch_types=sets + [
            pltpu.VMEM((NPAD,), jnp.float32),
            pltpu.VMEM((NPAD,), jnp.float32),
            pltpu.VMEM((ROWS_PT,), jnp.float32),
            pltpu.VMEM_SHARED((NPAD,), jnp.float32),
        ] + [pltpu.SemaphoreType.DMA] * 6,
        compiler_params=pltpu.CompilerParams(needs_layout_passes=False),
    )


def _msg_phase_body(senders, receivers, m_t, p_in, den, acc_out, *scr):
    s0 = SimpleNamespace(sidx=scr[0], ridx=scr[1], pbuf=scr[2], wbuf=scr[3],
                         rows_m=scr[4], semi=scr[13], semg=scr[14])
    s1 = SimpleNamespace(sidx=scr[5], ridx=scr[6], pbuf=scr[7], wbuf=scr[8],
                         rows_m=scr[9], semi=scr[15], semg=scr[16])
    dtab, dtmp, acc_sp = scr[10], scr[11], scr[12]

    cid = lax.axis_index("c")
    sid = lax.axis_index("s")
    wid = sid * NC + cid
    base0 = wid * EPW

    pltpu.sync_copy(den.at[0], dtab)
    pltpu.sync_copy(den.at[1], dtmp)

    def _d(i, _):
        sl = pl.ds(i * 16, 16)
        dtab[sl] = dtab[sl] + dtmp[sl]
        return 0
    lax.fori_loop(0, NPAD // 16, _d, 0)

    def _zrow(i, _):
        for k in range(D // 16):
            s0.rows_m[i, pl.ds(k * 16, 16)] = jnp.zeros((16,), jnp.float32)
        return 0
    lax.fori_loop(0, BE, _zrow, 0)
    for k in range(ROWS_PT // BE):
        pltpu.sync_copy(s0.rows_m, acc_sp.at[pl.ds(sid * ROWS_PT + k * BE, BE)])
    plsc.subcore_barrier()

    def idx_copies(j, S):
        base = base0 + j * BE
        return (
            pltpu.make_async_copy(senders.at[pl.ds(base, BE)], S.sidx, S.semi),
            pltpu.make_async_copy(receivers.at[pl.ds(base, BE)], S.ridx, S.semi),
            pltpu.make_async_copy(p_in.at[pl.ds(base, BE)], S.pbuf, S.semi),
        )

    def gather_copy(S):
        return pltpu.make_async_copy(m_t.at[S.sidx], S.rows_m, S.semg)

    def halfstep(j, sc, sn, next1, next2):
        if next1:
            for c in idx_copies(j + 1, sn):
                c.wait()
            gather_copy(sn).start()

        def _g(gi, _):
            sl = pl.ds(gi * 16, 16)
            dv = plsc.load_gather(dtab, [sc.ridx[sl]])
            sc.wbuf[sl] = sc.pbuf[sl] / dv
            return 0
        lax.fori_loop(0, BE // 16, _g, 0)

        gather_copy(sc).wait()

        def _row(i, _):
            wv = plsc.load_gather(sc.wbuf, [jnp.full((16,), i, jnp.int32)])
            for k in range(D // 16):
                sl = pl.ds(k * 16, 16)
                sc.rows_m[i, sl] = sc.rows_m[i, sl] * wv
            return 0
        lax.fori_loop(0, BE, _row, 0)

        pltpu.sync_copy(sc.rows_m, acc_sp.at[sc.ridx], add=True)
        if next2:
            for c in idx_copies(j + 2, sc):
                c.start()

    for c in idx_copies(0, s0):
        c.start()
    for c in idx_copies(0, s0):
        c.wait()
    gather_copy(s0).start()
    for c in idx_copies(1, s1):
        c.start()

    halfstep(0, s0, s1, True, True)

    def _pair(k, _):
        halfstep(2 * k + 1, s1, s0, True, True)
        halfstep(2 * k + 2, s0, s1, True, True)
        return 0
    lax.fori_loop(0, (NBLK - 3) // 2, _pair, 0)

    halfstep(NBLK - 2, s1, s0, True, False)
    halfstep(NBLK - 1, s0, s1, False, False)

    plsc.subcore_barrier()
    for k in range(ROWS_PT // BE):
        sl = pl.ds(sid * ROWS_PT + k * BE, BE)
        pltpu.sync_copy(acc_sp.at[sl], s0.rows_m)
        pltpu.sync_copy(s0.rows_m, acc_out.at[cid, sl])


@functools.cache
def _msg_phase():
    sets = []
    for _ in range(2):
        sets += [
            pltpu.VMEM((BE,), jnp.int32),
            pltpu.VMEM((BE,), jnp.int32),
            pltpu.VMEM((BE,), jnp.float32),
            pltpu.VMEM((BE,), jnp.float32),
            pltpu.VMEM((BE, D), jnp.float32),
        ]
    return pl.kernel(
        _msg_phase_body,
        out_type=jax.ShapeDtypeStruct((NC, NPAD, D), jnp.float32),
        mesh=_mesh(),
        scratch_types=sets + [
            pltpu.VMEM((NPAD,), jnp.float32),
            pltpu.VMEM((NPAD,), jnp.float32),
            pltpu.VMEM_SHARED((NPAD, D), jnp.float32),
        ] + [pltpu.SemaphoreType.DMA] * 4,
        compiler_params=pltpu.CompilerParams(needs_layout_passes=False),
    )


def kernel(nodes, edges, senders, receivers,
           W1, b1, W2, b2, W3, b3, W4, b4, W5, b5):
    s32 = senders.astype(jnp.int32)
    r32 = receivers.astype(jnp.int32)

    a_t, b_t, m_t, a_sc, b_sc = _node_mm(nodes, W1, b1, W2, b2, W5, b5, W4)
    ef0, e_sc = _edge_mm(edges, W3, b3, W4, b4)

    ef, p, den = _edge_phase()(
        s32, r32, a_t, b_t, ef0,
        a_sc.reshape(N), b_sc.reshape(N), e_sc.reshape(E))
    acc = _msg_phase()(s32, r32, m_t, p, den)
    new_nodes = _sum_cores(acc)[:N]
    return new_nodes, ef


# speedup vs baseline: 10.3496x; 1.1807x over previous
import functools
from types import SimpleNamespace

import jax
import jax.numpy as jnp
from jax import lax
from jax.experimental import pallas as pl
from jax.experimental.pallas import tpu as pltpu
from jax.experimental.pallas import tpu_sc as plsc

N = 10000
E = 320000
D = 128
DE = 16

NC = 2
NS = 16
NW = NC * NS
EPW = E // NW
BE = 80
NBLK = EPW // BE
NPAD = 10240
ROWS_PT = NPAD // NS


@functools.cache
def _mesh():
    return plsc.VectorSubcoreMesh(
        core_axis_name="c", subcore_axis_name="s",
        num_cores=NC, num_subcores=NS)


def _node_mm_body(x_ref, w1, b1, w2, b2, w5, b5, w4r,
                  a_out, b_out, m_out, as_out, bs_out):
    x = x_ref[...]
    a = jnp.dot(x, w1[...], preferred_element_type=jnp.float32) + b1[...]
    b = jnp.dot(x, w2[...], preferred_element_type=jnp.float32) + b2[...]
    m = jnp.dot(x, w5[...], preferred_element_type=jnp.float32) + b5[...]
    a_out[...] = a
    b_out[...] = b
    m_out[...] = m
    w4v = w4r[...]
    blk = x.shape[0]
    as_out[...] = jnp.sum(a * w4v, axis=1).reshape(1, 1, blk)
    bs_out[...] = jnp.sum(b * w4v, axis=1).reshape(1, 1, blk)


def _edge_mm_body(et_ref, w3, b3, w4r, b4, ef0_out, es_out):
    et = et_ref[...]
    ef0 = lax.dot_general(et, w3[...], (((0,), (0,)), ((), ())),
                          preferred_element_type=jnp.float32) + b3[...]
    ef0_out[...] = ef0
    blk = et.shape[1]
    es_out[...] = (jnp.sum(ef0 * w4r[...], axis=1) + b4[0, 0]).reshape(1, 1, blk)


def _sum_cores_body(acc_ref, out_ref):
    out_ref[...] = acc_ref[0] + acc_ref[1]


def _node_mm(nodes, W1, b1, W2, b2, W5, b5, W4):
    blk = 400
    grid = N // blk
    full = lambda shape: pl.BlockSpec(shape, lambda i: (0, 0))
    return pl.pallas_call(
        _node_mm_body,
        grid=(grid,),
        in_specs=[
            pl.BlockSpec((blk, D), lambda i: (i, 0)),
            full((D, D)), full((1, D)),
            full((D, D)), full((1, D)),
            full((D, D)), full((1, D)),
            full((1, D)),
        ],
        out_specs=[
            pl.BlockSpec((blk, D), lambda i: (i, 0)),
            pl.BlockSpec((blk, D), lambda i: (i, 0)),
            pl.BlockSpec((blk, D), lambda i: (i, 0)),
            pl.BlockSpec((1, 1, blk), lambda i: (i, 0, 0)),
            pl.BlockSpec((1, 1, blk), lambda i: (i, 0, 0)),
        ],
        out_shape=[
            jax.ShapeDtypeStruct((N, D), jnp.float32),
            jax.ShapeDtypeStruct((N, D), jnp.float32),
            jax.ShapeDtypeStruct((N, D), jnp.float32),
            jax.ShapeDtypeStruct((grid, 1, blk), jnp.float32),
            jax.ShapeDtypeStruct((grid, 1, blk), jnp.float32),
        ],
    )(nodes, W1, b1.reshape(1, D), W2, b2.reshape(1, D),
      W5, b5.reshape(1, D), W4.reshape(1, D))


def _edge_mm(edges, W3, b3, W4, b4):
    blk = 2560
    grid = E // blk
    full = lambda shape: pl.BlockSpec(shape, lambda i: (0, 0))
    return pl.pallas_call(
        _edge_mm_body,
        grid=(grid,),
        in_specs=[
            pl.BlockSpec((DE, blk), lambda i: (0, i)),
            full((DE, D)), full((1, D)),
            full((1, D)), full((1, 1)),
        ],
        out_specs=[
            pl.BlockSpec((blk, D), lambda i: (i, 0)),
            pl.BlockSpec((1, 1, blk), lambda i: (i, 0, 0)),
        ],
        out_shape=[
            jax.ShapeDtypeStruct((E, D), jnp.float32),
            jax.ShapeDtypeStruct((grid, 1, blk), jnp.float32),
        ],
    )(edges.T, W3, b3.reshape(1, D), W4.reshape(1, D), b4.reshape(1, 1))


def _sum_cores(acc):
    blk = 2048
    grid = NPAD // blk
    return pl.pallas_call(
        _sum_cores_body,
        grid=(grid,),
        in_specs=[pl.BlockSpec((NC, blk, D), lambda i: (0, i, 0))],
        out_specs=pl.BlockSpec((blk, D), lambda i: (i, 0)),
        out_shape=jax.ShapeDtypeStruct((NPAD, D), jnp.float32),
    )(acc)


def _edge_phase_body(senders, receivers, a_t, b_t, ef0, a_sc, b_sc, e_sc,
                     ef_out, p_out, den_out, *scr):
    s0 = SimpleNamespace(sidx=scr[0], ridx=scr[1], escb=scr[2], pbuf=scr[3],
                         rows_a=scr[4], rows_b=scr[5], ef0b=scr[6],
                         semi=scr[18], semg=scr[19], semw=scr[20])
    s1 = SimpleNamespace(sidx=scr[7], ridx=scr[8], escb=scr[9], pbuf=scr[10],
                         rows_a=scr[11], rows_b=scr[12], ef0b=scr[13],
                         semi=scr[21], semg=scr[22], semw=scr[23])
    atab, btab, dtile, dsp = scr[14], scr[15], scr[16], scr[17]

    cid = lax.axis_index("c")
    sid = lax.axis_index("s")
    wid = sid * NC + cid
    base0 = wid * EPW

    def _z(i, _):
        dtile[pl.ds(i * 16, 16)] = jnp.zeros((16,), jnp.float32)
        return 0
    lax.fori_loop(0, ROWS_PT // 16, _z, 0)
    pltpu.sync_copy(dtile, dsp.at[pl.ds(sid * ROWS_PT, ROWS_PT)])

    pltpu.sync_copy(a_sc, atab.at[pl.ds(0, N)])
    pltpu.sync_copy(b_sc, btab.at[pl.ds(0, N)])
    plsc.subcore_barrier()

    def idx_copies(j, S):
        base = base0 + j * BE
        return (
            pltpu.make_async_copy(senders.at[pl.ds(base, BE)], S.sidx, S.semi),
            pltpu.make_async_copy(receivers.at[pl.ds(base, BE)], S.ridx, S.semi),
            pltpu.make_async_copy(e_sc.at[pl.ds(base, BE)], S.escb, S.semi),
        )

    def gather_copies(j, S):
        base = base0 + j * BE
        return (
            pltpu.make_async_copy(a_t.at[S.sidx], S.rows_a, S.semg),
            pltpu.make_async_copy(b_t.at[S.ridx], S.rows_b, S.semg),
            pltpu.make_async_copy(ef0.at[pl.ds(base, BE)], S.ef0b, S.semg),
        )

    def ef_write(j, S):
        base = base0 + j * BE
        return pltpu.make_async_copy(S.ef0b, ef_out.at[pl.ds(base, BE)], S.semw)

    def halfstep(j, sc, sn, next1, next2, drainw):
        if next1:
            for c in idx_copies(j + 1, sn):
                c.wait()
            if drainw:
                ef_write(j - 1, sn).wait()
            for c in gather_copies(j + 1, sn):
                c.start()
        elif drainw:
            ef_write(j - 1, sn).wait()

        def _g(gi, _):
            sl = pl.ds(gi * 16, 16)
            av = plsc.load_gather(atab, [sc.sidx[sl]])
            bv = plsc.load_gather(btab, [sc.ridx[sl]])
            logit = av + bv + sc.escb[sl]
            logit = jnp.where(logit > 0, logit, logit * jnp.float32(0.01))
            sc.pbuf[sl] = jnp.exp(logit)
            return 0
        lax.fori_loop(0, BE // 16, _g, 0)

        for c in gather_copies(j, sc):
            c.wait()

        def _row(i, _):
            for k in range(D // 16):
                sl = pl.ds(k * 16, 16)
                sc.ef0b[i, sl] = sc.ef0b[i, sl] + sc.rows_a[i, sl] + sc.rows_b[i, sl]
            return 0
        lax.fori_loop(0, BE, _row, 0)

        ef_write(j, sc).start()
        base = base0 + j * BE
        pltpu.sync_copy(sc.pbuf, p_out.at[pl.ds(base, BE)])
        pltpu.sync_copy(sc.pbuf, dsp.at[sc.ridx], add=True)
        if next2:
            for c in idx_copies(j + 2, sc):
                c.start()

    for c in idx_copies(0, s0):
        c.start()
    for c in idx_copies(0, s0):
        c.wait()
    for c in gather_copies(0, s0):
        c.start()
    for c in idx_copies(1, s1):
        c.start()

    halfstep(0, s0, s1, True, True, False)

    def _pair(k, _):
        halfstep(2 * k + 1, s1, s0, True, True, True)
        halfstep(2 * k + 2, s0, s1, True, True, True)
        return 0
    lax.fori_loop(0, (NBLK - 3) // 2, _pair, 0)

    halfstep(NBLK - 2, s1, s0, True, False, True)
    halfstep(NBLK - 1, s0, s1, False, False, True)
    ef_write(NBLK - 1, s0).wait()

    plsc.subcore_barrier()
    pltpu.sync_copy(dsp.at[pl.ds(sid * ROWS_PT, ROWS_PT)], dtile)
    pltpu.sync_copy(dtile, den_out.at[cid, pl.ds(sid * ROWS_PT, ROWS_PT)])


@functools.cache
def _edge_phase():
    sets = []
    for _ in range(2):
        sets += [
            pltpu.VMEM((BE,), jnp.int32),
            pltpu.VMEM((BE,), jnp.int32),
            pltpu.VMEM((BE,), jnp.float32),
            pltpu.VMEM((BE,), jnp.float32),
            pltpu.VMEM((BE, D), jnp.float32),
            pltpu.VMEM((BE, D), jnp.float32),
            pltpu.VMEM((BE, D), jnp.float32),
        ]
    return pl.kernel(
        _edge_phase_body,
        out_type=(
            jax.ShapeDtypeStruct((E, D), jnp.float32),
            jax.ShapeDtypeStruct((E,), jnp.float32),
            jax.ShapeDtypeStruct((NC, NPAD), jnp.float32),
        ),
        mesh=_mesh(),
        scratch_types=sets + [
            pltpu.VMEM((NPAD,), jnp.float32),
            pltpu.VMEM((NPAD,), jnp.float32),
            pltpu.VMEM((ROWS_PT,), jnp.float32),
            pltpu.VMEM_SHARED((NPAD,), jnp.float32),
        ] + [pltpu.SemaphoreType.DMA] * 6,
        compiler_params=pltpu.CompilerParams(needs_layout_passes=False),
    )


def _msg_phase_body(senders, receivers, m_t, p_in, den, acc_out, *scr):
    s0 = SimpleNamespace(sidx=scr[0], ridx=scr[1], pbuf=scr[2], wbuf=scr[3],
                         rows_m=scr[4], semi=scr[13], semg=scr[14])
    s1 = SimpleNamespace(sidx=scr[5], ridx=scr[6], pbuf=scr[7], wbuf=scr[8],
                         rows_m=scr[9], semi=scr[15], semg=scr[16])
    dtab, dtmp, acc_sp = scr[10], scr[11], scr[12]

    cid = lax.axis_index("c")
    sid = lax.axis_index("s")
    wid = sid * NC + cid
    base0 = wid * EPW

    pltpu.sync_copy(den.at[0], dtab)
    pltpu.sync_copy(den.at[1], dtmp)

    def _d(i, _):
        sl = pl.ds(i * 16, 16)
        dtab[sl] = dtab[sl] + dtmp[sl]
        return 0
    lax.fori_loop(0, NPAD // 16, _d, 0)

    def _zrow(i, _):
        for k in range(D // 16):
            s0.rows_m[i, pl.ds(k * 16, 16)] = jnp.zeros((16,), jnp.float32)
        return 0
    lax.fori_loop(0, BE, _zrow, 0)
    for k in range(ROWS_PT // BE):
        pltpu.sync_copy(s0.rows_m, acc_sp.at[pl.ds(sid * ROWS_PT + k * BE, BE)])
    plsc.subcore_barrier()

    def idx_copies(j, S):
        base = base0 + j * BE
        return (
            pltpu.make_async_copy(senders.at[pl.ds(base, BE)], S.sidx, S.semi),
            pltpu.make_async_copy(receivers.at[pl.ds(base, BE)], S.ridx, S.semi),
            pltpu.make_async_copy(p_in.at[pl.ds(base, BE)], S.pbuf, S.semi),
        )

    def gather_copy(S):
        return pltpu.make_async_copy(m_t.at[S.sidx], S.rows_m, S.semg)

    def halfstep(j, sc, sn, next1, next2):
        if next1:
            for c in idx_copies(j + 1, sn):
                c.wait()
            gather_copy(sn).start()

        def _g(gi, _):
            sl = pl.ds(gi * 16, 16)
            dv = plsc.load_gather(dtab, [sc.ridx[sl]])
            sc.wbuf[sl] = sc.pbuf[sl] / dv
            return 0
        lax.fori_loop(0, BE // 16, _g, 0)

        gather_copy(sc).wait()

        def _row(i, _):
            wv = plsc.load_gather(sc.wbuf, [jnp.full((16,), i, jnp.int32)])
            for k in range(D // 16):
                sl = pl.ds(k * 16, 16)
                sc.rows_m[i, sl] = sc.rows_m[i, sl] * wv
            return 0
        lax.fori_loop(0, BE, _row, 0)

        pltpu.sync_copy(sc.rows_m, acc_sp.at[sc.ridx], add=True)
        if next2:
            for c in idx_copies(j + 2, sc):
                c.start()

    for c in idx_copies(0, s0):
        c.start()
    for c in idx_copies(0, s0):
        c.wait()
    gather_copy(s0).start()
    for c in idx_copies(1, s1):
        c.start()

    halfstep(0, s0, s1, True, True)

    def _pair(k, _):
        halfstep(2 * k + 1, s1, s0, True, True)
        halfstep(2 * k + 2, s0, s1, True, True)
        return 0
    lax.fori_loop(0, (NBLK - 3) // 2, _pair, 0)

    halfstep(NBLK - 2, s1, s0, True, False)
    halfstep(NBLK - 1, s0, s1, False, False)

    plsc.subcore_barrier()
    for k in range(ROWS_PT // BE):
        sl = pl.ds(sid * ROWS_PT + k * BE, BE)
        pltpu.sync_copy(acc_sp.at[sl], s0.rows_m)
        pltpu.sync_copy(s0.rows_m, acc_out.at[cid, sl])


@functools.cache
def _msg_phase():
    sets = []
    for _ in range(2):
        sets += [
            pltpu.VMEM((BE,), jnp.int32),
            pltpu.VMEM((BE,), jnp.int32),
            pltpu.VMEM((BE,), jnp.float32),
            pltpu.VMEM((BE,), jnp.float32),
            pltpu.VMEM((BE, D), jnp.float32),
        ]
    return pl.kernel(
        _msg_phase_body,
        out_type=jax.ShapeDtypeStruct((NC, NPAD, D), jnp.float32),
        mesh=_mesh(),
        scratch_types=sets + [
            pltpu.VMEM((NPAD,), jnp.float32),
            pltpu.VMEM((NPAD,), jnp.float32),
            pltpu.VMEM_SHARED((NPAD, D), jnp.float32),
        ] + [pltpu.SemaphoreType.DMA] * 4,
        compiler_params=pltpu.CompilerParams(needs_layout_passes=False),
    )


def kernel(nodes, edges, senders, receivers,
           W1, b1, W2, b2, W3, b3, W4, b4, W5, b5):
    s32 = senders.astype(jnp.int32)
    r32 = receivers.astype(jnp.int32)

    a_t, b_t, m_t, a_sc, b_sc = _node_mm(nodes, W1, b1, W2, b2, W5, b5, W4)
    ef0, e_sc = _edge_mm(edges, W3, b3, W4, b4)

    ef, p, den = _edge_phase()(
        s32, r32, a_t, b_t, ef0,
        a_sc.reshape(N), b_sc.reshape(N), e_sc.reshape(E))
    acc = _msg_phase()(s32, r32, m_t, p, den)
    new_nodes = _sum_cores(acc)[:N]
    return new_nodes, ef


# speedup vs baseline: 11.6071x; 1.1215x over previous
import functools
from types import SimpleNamespace

import jax
import jax.numpy as jnp
from jax import lax
from jax.experimental import pallas as pl
from jax.experimental.pallas import tpu as pltpu
from jax.experimental.pallas import tpu_sc as plsc

N = 10000
E = 320000
D = 128
DE = 16

NC = 2
NS = 16
NW = NC * NS
EPW = E // NW
BE = 80
NBLK = EPW // BE
NPAD = 10240
ROWS_PT = NPAD // NS


@functools.cache
def _mesh():
    return plsc.VectorSubcoreMesh(
        core_axis_name="c", subcore_axis_name="s",
        num_cores=NC, num_subcores=NS)


def _node_mm_body(x_ref, w1, b1, w2, b2, w5, b5, w4,
                  a_out, b_out, m_out, as_out, bs_out):
    x = x_ref[...]
    a = jnp.dot(x, w1[...], preferred_element_type=jnp.float32) + b1[...]
    b = jnp.dot(x, w2[...], preferred_element_type=jnp.float32) + b2[...]
    m = jnp.dot(x, w5[...], preferred_element_type=jnp.float32) + b5[...]
    a_out[...] = a
    b_out[...] = b
    m_out[...] = m
    w4v = w4[...]
    blk = x.shape[0]
    as_out[...] = lax.dot_general(w4v, a, (((0,), (1,)), ((), ())),
                                  preferred_element_type=jnp.float32
                                  ).reshape(1, 1, blk)
    bs_out[...] = lax.dot_general(w4v, b, (((0,), (1,)), ((), ())),
                                  preferred_element_type=jnp.float32
                                  ).reshape(1, 1, blk)


def _edge_mm_body(et_ref, w3, b3, w4, b4, ef0_out, es_out):
    et = et_ref[...]
    ef0 = lax.dot_general(et, w3[...], (((0,), (0,)), ((), ())),
                          preferred_element_type=jnp.float32) + b3[...]
    ef0_out[...] = ef0
    blk = et.shape[1]
    w34 = jnp.dot(w3[...], w4[...], preferred_element_type=jnp.float32)
    c = jnp.dot(b3[...], w4[...], preferred_element_type=jnp.float32)
    es = lax.dot_general(w34, et, (((0,), (0,)), ((), ())),
                         preferred_element_type=jnp.float32)
    es_out[...] = (es + c[0, 0] + b4[0, 0]).reshape(1, 1, blk)


def _sum_cores_body(acc_ref, out_ref):
    out_ref[...] = acc_ref[0] + acc_ref[1]


def _node_mm(nodes, W1, b1, W2, b2, W5, b5, W4):
    blk = 400
    grid = N // blk
    full = lambda shape: pl.BlockSpec(shape, lambda i: (0, 0))
    return pl.pallas_call(
        _node_mm_body,
        grid=(grid,),
        in_specs=[
            pl.BlockSpec((blk, D), lambda i: (i, 0)),
            full((D, D)), full((1, D)),
            full((D, D)), full((1, D)),
            full((D, D)), full((1, D)),
            full((D, 1)),
        ],
        out_specs=[
            pl.BlockSpec((blk, D), lambda i: (i, 0)),
            pl.BlockSpec((blk, D), lambda i: (i, 0)),
            pl.BlockSpec((blk, D), lambda i: (i, 0)),
            pl.BlockSpec((1, 1, blk), lambda i: (i, 0, 0)),
            pl.BlockSpec((1, 1, blk), lambda i: (i, 0, 0)),
        ],
        out_shape=[
            jax.ShapeDtypeStruct((N, D), jnp.float32),
            jax.ShapeDtypeStruct((N, D), jnp.float32),
            jax.ShapeDtypeStruct((N, D), jnp.float32),
            jax.ShapeDtypeStruct((grid, 1, blk), jnp.float32),
            jax.ShapeDtypeStruct((grid, 1, blk), jnp.float32),
        ],
    )(nodes, W1, b1.reshape(1, D), W2, b2.reshape(1, D),
      W5, b5.reshape(1, D), W4)


def _edge_mm(edges, W3, b3, W4, b4):
    blk = 2560
    grid = E // blk
    full = lambda shape: pl.BlockSpec(shape, lambda i: (0, 0))
    return pl.pallas_call(
        _edge_mm_body,
        grid=(grid,),
        in_specs=[
            pl.BlockSpec((DE, blk), lambda i: (0, i)),
            full((DE, D)), full((1, D)),
            full((D, 1)), full((1, 1)),
        ],
        out_specs=[
            pl.BlockSpec((blk, D), lambda i: (i, 0)),
            pl.BlockSpec((1, 1, blk), lambda i: (i, 0, 0)),
        ],
        out_shape=[
            jax.ShapeDtypeStruct((E, D), jnp.float32),
            jax.ShapeDtypeStruct((grid, 1, blk), jnp.float32),
        ],
    )(edges.T, W3, b3.reshape(1, D), W4, b4.reshape(1, 1))


def _sum_cores(acc):
    blk = 2048
    grid = NPAD // blk
    return pl.pallas_call(
        _sum_cores_body,
        grid=(grid,),
        in_specs=[pl.BlockSpec((NC, blk, D), lambda i: (0, i, 0))],
        out_specs=pl.BlockSpec((blk, D), lambda i: (i, 0)),
        out_shape=jax.ShapeDtypeStruct((NPAD, D), jnp.float32),
    )(acc)


def _edge_phase_body(senders, receivers, a_t, b_t, ef0, a_sc, b_sc, e_sc,
                     ef_out, p_out, den_out, *scr):
    s0 = SimpleNamespace(sidx=scr[0], ridx=scr[1], escb=scr[2], pbuf=scr[3],
                         rows_a=scr[4], rows_b=scr[5], ef0b=scr[6],
                         semi=scr[18], semg=scr[19], semw=scr[20])
    s1 = SimpleNamespace(sidx=scr[7], ridx=scr[8], escb=scr[9], pbuf=scr[10],
                         rows_a=scr[11], rows_b=scr[12], ef0b=scr[13],
                         semi=scr[21], semg=scr[22], semw=scr[23])
    atab, btab, dtile, dsp = scr[14], scr[15], scr[16], scr[17]

    cid = lax.axis_index("c")
    sid = lax.axis_index("s")
    wid = sid * NC + cid
    base0 = wid * EPW

    def _z(i, _):
        dtile[pl.ds(i * 16, 16)] = jnp.zeros((16,), jnp.float32)
        return 0
    lax.fori_loop(0, ROWS_PT // 16, _z, 0)
    pltpu.sync_copy(dtile, dsp.at[pl.ds(sid * ROWS_PT, ROWS_PT)])

    pltpu.sync_copy(a_sc, atab.at[pl.ds(0, N)])
    pltpu.sync_copy(b_sc, btab.at[pl.ds(0, N)])
    plsc.subcore_barrier()

    def idx_copies(j, S):
        base = base0 + j * BE
        return (
            pltpu.make_async_copy(senders.at[pl.ds(base, BE)], S.sidx, S.semi),
            pltpu.make_async_copy(receivers.at[pl.ds(base, BE)], S.ridx, S.semi),
            pltpu.make_async_copy(e_sc.at[pl.ds(base, BE)], S.escb, S.semi),
        )

    def gather_copies(j, S):
        base = base0 + j * BE
        return (
            pltpu.make_async_copy(a_t.at[S.sidx], S.rows_a, S.semg),
            pltpu.make_async_copy(b_t.at[S.ridx], S.rows_b, S.semg),
            pltpu.make_async_copy(ef0.at[pl.ds(base, BE)], S.ef0b, S.semg),
        )

    def ef_write(j, S):
        base = base0 + j * BE
        return pltpu.make_async_copy(S.ef0b, ef_out.at[pl.ds(base, BE)], S.semw)

    def halfstep(j, sc, sn, next1, next2, drainw):
        if next1:
            for c in idx_copies(j + 1, sn):
                c.wait()
            if drainw:
                ef_write(j - 1, sn).wait()
            for c in gather_copies(j + 1, sn):
                c.start()
        elif drainw:
            ef_write(j - 1, sn).wait()

        def _g(gi, _):
            sl = pl.ds(gi * 16, 16)
            av = plsc.load_gather(atab, [sc.sidx[sl]])
            bv = plsc.load_gather(btab, [sc.ridx[sl]])
            logit = av + bv + sc.escb[sl]
            logit = jnp.where(logit > 0, logit, logit * jnp.float32(0.01))
            sc.pbuf[sl] = jnp.exp(logit)
            return 0
        lax.fori_loop(0, BE // 16, _g, 0)

        for c in gather_copies(j, sc):
            c.wait()

        def _row(i, _):
            for k in range(D // 16):
                sl = pl.ds(k * 16, 16)
                sc.ef0b[i, sl] = sc.ef0b[i, sl] + sc.rows_a[i, sl] + sc.rows_b[i, sl]
            return 0
        lax.fori_loop(0, BE, _row, 0)

        ef_write(j, sc).start()
        base = base0 + j * BE
        pltpu.sync_copy(sc.pbuf, p_out.at[pl.ds(base, BE)])
        pltpu.sync_copy(sc.pbuf, dsp.at[sc.ridx], add=True)
        if next2:
            for c in idx_copies(j + 2, sc):
                c.start()

    for c in idx_copies(0, s0):
        c.start()
    for c in idx_copies(0, s0):
        c.wait()
    for c in gather_copies(0, s0):
        c.start()
    for c in idx_copies(1, s1):
        c.start()

    halfstep(0, s0, s1, True, True, False)

    def _pair(k, _):
        halfstep(2 * k + 1, s1, s0, True, True, True)
        halfstep(2 * k + 2, s0, s1, True, True, True)
        return 0
    lax.fori_loop(0, (NBLK - 3) // 2, _pair, 0)

    halfstep(NBLK - 2, s1, s0, True, False, True)
    halfstep(NBLK - 1, s0, s1, False, False, True)
    ef_write(NBLK - 1, s0).wait()

    plsc.subcore_barrier()
    pltpu.sync_copy(dsp.at[pl.ds(sid * ROWS_PT, ROWS_PT)], dtile)
    pltpu.sync_copy(dtile, den_out.at[cid, pl.ds(sid * ROWS_PT, ROWS_PT)])


@functools.cache
def _edge_phase():
    sets = []
    for _ in range(2):
        sets += [
            pltpu.VMEM((BE,), jnp.int32),
            pltpu.VMEM((BE,), jnp.int32),
            pltpu.VMEM((BE,), jnp.float32),
            pltpu.VMEM((BE,), jnp.float32),
            pltpu.VMEM((BE, D), jnp.float32),
            pltpu.VMEM((BE, D), jnp.float32),
            pltpu.VMEM((BE, D), jnp.float32),
        ]
    return pl.kernel(
        _edge_phase_body,
        out_type=(
            jax.ShapeDtypeStruct((E, D), jnp.float32),
            jax.ShapeDtypeStruct((E,), jnp.float32),
            jax.ShapeDtypeStruct((NC, NPAD), jnp.float32),
        ),
        mesh=_mesh(),
        scratch_types=sets + [
            pltpu.VMEM((NPAD,), jnp.float32),
            pltpu.VMEM((NPAD,), jnp.float32),
            pltpu.VMEM((ROWS_PT,), jnp.float32),
            pltpu.VMEM_SHARED((NPAD,), jnp.float32),
        ] + [pltpu.SemaphoreType.DMA] * 6,
        compiler_params=pltpu.CompilerParams(needs_layout_passes=False),
    )


def _msg_phase_body(senders, receivers, m_t, p_in, den, acc_out, *scr):
    s0 = SimpleNamespace(sidx=scr[0], ridx=scr[1], pbuf=scr[2], wbuf=scr[3],
                         rows_m=scr[4], semi=scr[13], semg=scr[14])
    s1 = SimpleNamespace(sidx=scr[5], ridx=scr[6], pbuf=scr[7], wbuf=scr[8],
                         rows_m=scr[9], semi=scr[15], semg=scr[16])
    dtab, dtmp, acc_sp = scr[10], scr[11], scr[12]

    cid = lax.axis_index("c")
    sid = lax.axis_index("s")
    wid = sid * NC + cid
    base0 = wid * EPW

    pltpu.sync_copy(den.at[0], dtab)
    pltpu.sync_copy(den.at[1], dtmp)

    def _d(i, _):
        sl = pl.ds(i * 16, 16)
        dtab[sl] = dtab[sl] + dtmp[sl]
        return 0
    lax.fori_loop(0, NPAD // 16, _d, 0)

    def _zrow(i, _):
        for k in range(D // 16):
            s0.rows_m[i, pl.ds(k * 16, 16)] = jnp.zeros((16,), jnp.float32)
        return 0
    lax.fori_loop(0, BE, _zrow, 0)
    for k in range(ROWS_PT // BE):
        pltpu.sync_copy(s0.rows_m, acc_sp.at[pl.ds(sid * ROWS_PT + k * BE, BE)])
    plsc.subcore_barrier()

    def idx_copies(j, S):
        base = base0 + j * BE
        return (
            pltpu.make_async_copy(senders.at[pl.ds(base, BE)], S.sidx, S.semi),
            pltpu.make_async_copy(receivers.at[pl.ds(base, BE)], S.ridx, S.semi),
            pltpu.make_async_copy(p_in.at[pl.ds(base, BE)], S.pbuf, S.semi),
        )

    def gather_copy(S):
        return pltpu.make_async_copy(m_t.at[S.sidx], S.rows_m, S.semg)

    def halfstep(j, sc, sn, next1, next2):
        if next1:
            for c in idx_copies(j + 1, sn):
                c.wait()
            gather_copy(sn).start()

        def _g(gi, _):
            sl = pl.ds(gi * 16, 16)
            dv = plsc.load_gather(dtab, [sc.ridx[sl]])
            sc.wbuf[sl] = sc.pbuf[sl] / dv
            return 0
        lax.fori_loop(0, BE // 16, _g, 0)

        gather_copy(sc).wait()

        def _row(i, _):
            wv = plsc.load_gather(sc.wbuf, [jnp.full((16,), i, jnp.int32)])
            for k in range(D // 16):
                sl = pl.ds(k * 16, 16)
                sc.rows_m[i, sl] = sc.rows_m[i, sl] * wv
            return 0
        lax.fori_loop(0, BE, _row, 0)

        pltpu.sync_copy(sc.rows_m, acc_sp.at[sc.ridx], add=True)
        if next2:
            for c in idx_copies(j + 2, sc):
                c.start()

    for c in idx_copies(0, s0):
        c.start()
    for c in idx_copies(0, s0):
        c.wait()
    gather_copy(s0).start()
    for c in idx_copies(1, s1):
        c.start()

    halfstep(0, s0, s1, True, True)

    def _pair(k, _):
        halfstep(2 * k + 1, s1, s0, True, True)
        halfstep(2 * k + 2, s0, s1, True, True)
        return 0
    lax.fori_loop(0, (NBLK - 3) // 2, _pair, 0)

    halfstep(NBLK - 2, s1, s0, True, False)
    halfstep(NBLK - 1, s0, s1, False, False)

    plsc.subcore_barrier()
    for k in range(ROWS_PT // BE):
        sl = pl.ds(sid * ROWS_PT + k * BE, BE)
        pltpu.sync_copy(acc_sp.at[sl], s0.rows_m)
        pltpu.sync_copy(s0.rows_m, acc_out.at[cid, sl])


@functools.cache
def _msg_phase():
    sets = []
    for _ in range(2):
        sets += [
            pltpu.VMEM((BE,), jnp.int32),
            pltpu.VMEM((BE,), jnp.int32),
            pltpu.VMEM((BE,), jnp.float32),
            pltpu.VMEM((BE,), jnp.float32),
            pltpu.VMEM((BE, D), jnp.float32),
        ]
    return pl.kernel(
        _msg_phase_body,
        out_type=jax.ShapeDtypeStruct((NC, NPAD, D), jnp.float32),
        mesh=_mesh(),
        scratch_types=sets + [
            pltpu.VMEM((NPAD,), jnp.float32),
            pltpu.VMEM((NPAD,), jnp.float32),
            pltpu.VMEM_SHARED((NPAD, D), jnp.float32),
        ] + [pltpu.SemaphoreType.DMA] * 4,
        compiler_params=pltpu.CompilerParams(needs_layout_passes=False),
    )


def kernel(nodes, edges, senders, receivers,
           W1, b1, W2, b2, W3, b3, W4, b4, W5, b5):
    s32 = senders.astype(jnp.int32)
    r32 = receivers.astype(jnp.int32)

    a_t, b_t, m_t, a_sc, b_sc = _node_mm(nodes, W1, b1, W2, b2, W5, b5, W4)
    ef0, e_sc = _edge_mm(edges, W3, b3, W4, b4)

    ef, p, den = _edge_phase()(
        s32, r32, a_t, b_t, ef0,
        a_sc.reshape(N), b_sc.reshape(N), e_sc.reshape(E))
    acc = _msg_phase()(s32, r32, m_t, p, den)
    new_nodes = _sum_cores(acc)[:N]
    return new_nodes, ef


# speedup vs baseline: 12.8225x; 1.1047x over previous
import functools
from types import SimpleNamespace

import jax
import jax.numpy as jnp
from jax import lax
from jax.experimental import pallas as pl
from jax.experimental.pallas import tpu as pltpu
from jax.experimental.pallas import tpu_sc as plsc

N = 10000
E = 320000
D = 128
DE = 16

NC = 2
NS = 16
NW = NC * NS
EPW = E // NW
BE = 80
NBLK = EPW // BE
NPAD = 10240
ROWS_PT = NPAD // NS


@functools.cache
def _mesh():
    return plsc.VectorSubcoreMesh(
        core_axis_name="c", subcore_axis_name="s",
        num_cores=NC, num_subcores=NS)


def _node_mm_body(x_ref, w1, b1, w2, b2, w5, b5, w4,
                  a_out, b_out, m_out, as_out, bs_out):
    x = x_ref[...]
    a = jnp.dot(x, w1[...], preferred_element_type=jnp.float32) + b1[...]
    b = jnp.dot(x, w2[...], preferred_element_type=jnp.float32) + b2[...]
    m = jnp.dot(x, w5[...], preferred_element_type=jnp.float32) + b5[...]
    a_out[...] = a
    b_out[...] = b
    m_out[...] = m
    w4v = w4[...]
    blk = x.shape[0]
    as_out[...] = lax.dot_general(w4v, a, (((0,), (1,)), ((), ())),
                                  preferred_element_type=jnp.float32
                                  ).reshape(1, 1, blk)
    bs_out[...] = lax.dot_general(w4v, b, (((0,), (1,)), ((), ())),
                                  preferred_element_type=jnp.float32
                                  ).reshape(1, 1, blk)


def _escal_mm_body(et_ref, w3, b3, w4, b4, es_out):
    et = et_ref[...]
    blk = et.shape[1]
    w34 = jnp.dot(w3[...], w4[...], preferred_element_type=jnp.float32)
    c = jnp.dot(b3[...], w4[...], preferred_element_type=jnp.float32)
    es = lax.dot_general(w34, et, (((0,), (0,)), ((), ())),
                         preferred_element_type=jnp.float32)
    es_out[...] = (es + c[0, 0] + b4[0, 0]).reshape(1, 1, blk)


def _ef_mm_body(g_ref, et_ref, w3, b3, ef_out):
    et = et_ref[...]
    ef_out[...] = g_ref[...] + b3[...] + lax.dot_general(
        et, w3[...], (((0,), (0,)), ((), ())),
        preferred_element_type=jnp.float32)


def _sum_cores_body(acc_ref, out_ref):
    out_ref[...] = acc_ref[0] + acc_ref[1]


def _node_mm(nodes, W1, b1, W2, b2, W5, b5, W4):
    blk = 400
    grid = N // blk
    full = lambda shape: pl.BlockSpec(shape, lambda i: (0, 0))
    return pl.pallas_call(
        _node_mm_body,
        grid=(grid,),
        in_specs=[
            pl.BlockSpec((blk, D), lambda i: (i, 0)),
            full((D, D)), full((1, D)),
            full((D, D)), full((1, D)),
            full((D, D)), full((1, D)),
            full((D, 1)),
        ],
        out_specs=[
            pl.BlockSpec((blk, D), lambda i: (i, 0)),
            pl.BlockSpec((blk, D), lambda i: (i, 0)),
            pl.BlockSpec((blk, D), lambda i: (i, 0)),
            pl.BlockSpec((1, 1, blk), lambda i: (i, 0, 0)),
            pl.BlockSpec((1, 1, blk), lambda i: (i, 0, 0)),
        ],
        out_shape=[
            jax.ShapeDtypeStruct((N, D), jnp.float32),
            jax.ShapeDtypeStruct((N, D), jnp.float32),
            jax.ShapeDtypeStruct((N, D), jnp.float32),
            jax.ShapeDtypeStruct((grid, 1, blk), jnp.float32),
            jax.ShapeDtypeStruct((grid, 1, blk), jnp.float32),
        ],
    )(nodes, W1, b1.reshape(1, D), W2, b2.reshape(1, D),
      W5, b5.reshape(1, D), W4)


def _escal_mm(edges_t, W3, b3, W4, b4):
    blk = 2560
    grid = E // blk
    full = lambda shape: pl.BlockSpec(shape, lambda i: (0, 0))
    return pl.pallas_call(
        _escal_mm_body,
        grid=(grid,),
        in_specs=[
            pl.BlockSpec((DE, blk), lambda i: (0, i)),
            full((DE, D)), full((1, D)),
            full((D, 1)), full((1, 1)),
        ],
        out_specs=pl.BlockSpec((1, 1, blk), lambda i: (i, 0, 0)),
        out_shape=jax.ShapeDtypeStruct((grid, 1, blk), jnp.float32),
    )(edges_t, W3, b3.reshape(1, D), W4, b4.reshape(1, 1))


def _ef_mm(g, edges_t, W3, b3):
    blk = 2560
    grid = E // blk
    full = lambda shape: pl.BlockSpec(shape, lambda i: (0, 0))
    return pl.pallas_call(
        _ef_mm_body,
        grid=(grid,),
        in_specs=[
            pl.BlockSpec((blk, D), lambda i: (i, 0)),
            pl.BlockSpec((DE, blk), lambda i: (0, i)),
            full((DE, D)), full((1, D)),
        ],
        out_specs=pl.BlockSpec((blk, D), lambda i: (i, 0)),
        out_shape=jax.ShapeDtypeStruct((E, D), jnp.float32),
    )(g, edges_t, W3, b3.reshape(1, D))


def _sum_cores(acc):
    blk = 2048
    grid = NPAD // blk
    return pl.pallas_call(
        _sum_cores_body,
        grid=(grid,),
        in_specs=[pl.BlockSpec((NC, blk, D), lambda i: (0, i, 0))],
        out_specs=pl.BlockSpec((blk, D), lambda i: (i, 0)),
        out_shape=jax.ShapeDtypeStruct((NPAD, D), jnp.float32),
    )(acc)


def _edge_phase_body(senders, receivers, a_t, b_t, a_sc, b_sc, e_sc,
                     g_out, p_out, den_out, *scr):
    s0 = SimpleNamespace(sidx=scr[0], ridx=scr[1], escb=scr[2], pbuf=scr[3],
                         rows_a=scr[4], rows_b=scr[5],
                         semi=scr[16], semg=scr[17], semw=scr[18])
    s1 = SimpleNamespace(sidx=scr[6], ridx=scr[7], escb=scr[8], pbuf=scr[9],
                         rows_a=scr[10], rows_b=scr[11],
                         semi=scr[19], semg=scr[20], semw=scr[21])
    atab, btab, dtile, dsp = scr[12], scr[13], scr[14], scr[15]

    cid = lax.axis_index("c")
    sid = lax.axis_index("s")
    wid = sid * NC + cid
    base0 = wid * EPW

    def _z(i, _):
        dtile[pl.ds(i * 16, 16)] = jnp.zeros((16,), jnp.float32)
        return 0
    lax.fori_loop(0, ROWS_PT // 16, _z, 0)
    pltpu.sync_copy(dtile, dsp.at[pl.ds(sid * ROWS_PT, ROWS_PT)])

    pltpu.sync_copy(a_sc, atab.at[pl.ds(0, N)])
    pltpu.sync_copy(b_sc, btab.at[pl.ds(0, N)])
    plsc.subcore_barrier()

    def idx_copies(j, S):
        base = base0 + j * BE
        return (
            pltpu.make_async_copy(senders.at[pl.ds(base, BE)], S.sidx, S.semi),
            pltpu.make_async_copy(receivers.at[pl.ds(base, BE)], S.ridx, S.semi),
            pltpu.make_async_copy(e_sc.at[pl.ds(base, BE)], S.escb, S.semi),
        )

    def gather_copies(j, S):
        return (
            pltpu.make_async_copy(a_t.at[S.sidx], S.rows_a, S.semg),
            pltpu.make_async_copy(b_t.at[S.ridx], S.rows_b, S.semg),
        )

    def g_write(j, S):
        base = base0 + j * BE
        return pltpu.make_async_copy(S.rows_a, g_out.at[pl.ds(base, BE)], S.semw)

    def halfstep(j, sc, sn, next1, next2, drainw):
        if next1:
            for c in idx_copies(j + 1, sn):
                c.wait()
            if drainw:
                g_write(j - 1, sn).wait()
            for c in gather_copies(j + 1, sn):
                c.start()
        elif drainw:
            g_write(j - 1, sn).wait()

        def _g(gi, _):
            sl = pl.ds(gi * 16, 16)
            av = plsc.load_gather(atab, [sc.sidx[sl]])
            bv = plsc.load_gather(btab, [sc.ridx[sl]])
            logit = av + bv + sc.escb[sl]
            logit = jnp.where(logit > 0, logit, logit * jnp.float32(0.01))
            sc.pbuf[sl] = jnp.exp(logit)
            return 0
        lax.fori_loop(0, BE // 16, _g, 0)

        for c in gather_copies(j, sc):
            c.wait()

        def _row(i, _):
            for k in range(D // 16):
                sl = pl.ds(k * 16, 16)
                sc.rows_a[i, sl] = sc.rows_a[i, sl] + sc.rows_b[i, sl]
            return 0
        lax.fori_loop(0, BE, _row, 0)

        g_write(j, sc).start()
        base = base0 + j * BE
        pltpu.sync_copy(sc.pbuf, p_out.at[pl.ds(base, BE)])
        pltpu.sync_copy(sc.pbuf, dsp.at[sc.ridx], add=True)
        if next2:
            for c in idx_copies(j + 2, sc):
                c.start()

    for c in idx_copies(0, s0):
        c.start()
    for c in idx_copies(0, s0):
        c.wait()
    for c in gather_copies(0, s0):
        c.start()
    for c in idx_copies(1, s1):
        c.start()

    halfstep(0, s0, s1, True, True, False)

    def _pair(k, _):
        halfstep(2 * k + 1, s1, s0, True, True, True)
        halfstep(2 * k + 2, s0, s1, True, True, True)
        return 0
    lax.fori_loop(0, (NBLK - 3) // 2, _pair, 0)

    halfstep(NBLK - 2, s1, s0, True, False, True)
    halfstep(NBLK - 1, s0, s1, False, False, True)
    g_write(NBLK - 1, s0).wait()

    plsc.subcore_barrier()
    pltpu.sync_copy(dsp.at[pl.ds(sid * ROWS_PT, ROWS_PT)], dtile)
    pltpu.sync_copy(dtile, den_out.at[cid, pl.ds(sid * ROWS_PT, ROWS_PT)])


@functools.cache
def _edge_phase():
    sets = []
    for _ in range(2):
        sets += [
            pltpu.VMEM((BE,), jnp.int32),
            pltpu.VMEM((BE,), jnp.int32),
            pltpu.VMEM((BE,), jnp.float32),
            pltpu.VMEM((BE,), jnp.float32),
            pltpu.VMEM((BE, D), jnp.float32),
            pltpu.VMEM((BE, D), jnp.float32),
        ]
    return pl.kernel(
        _edge_phase_body,
        out_type=(
            jax.ShapeDtypeStruct((E, D), jnp.float32),
            jax.ShapeDtypeStruct((E,), jnp.float32),
            jax.ShapeDtypeStruct((NC, NPAD), jnp.float32),
        ),
        mesh=_mesh(),
        scratch_types=sets + [
            pltpu.VMEM((NPAD,), jnp.float32),
            pltpu.VMEM((NPAD,), jnp.float32),
            pltpu.VMEM((ROWS_PT,), jnp.float32),
            pltpu.VMEM_SHARED((NPAD,), jnp.float32),
        ] + [pltpu.SemaphoreType.DMA] * 6,
        compiler_params=pltpu.CompilerParams(needs_layout_passes=False),
    )


def _msg_phase_body(senders, receivers, m_t, p_in, den, acc_out, *scr):
    s0 = SimpleNamespace(sidx=scr[0], ridx=scr[1], pbuf=scr[2], wbuf=scr[3],
                         rows_m=scr[4], semi=scr[13], semg=scr[14])
    s1 = SimpleNamespace(sidx=scr[5], ridx=scr[6], pbuf=scr[7], wbuf=scr[8],
                         rows_m=scr[9], semi=scr[15], semg=scr[16])
    dtab, dtmp, acc_sp = scr[10], scr[11], scr[12]

    cid = lax.axis_index("c")
    sid = lax.axis_index("s")
    wid = sid * NC + cid
    base0 = wid * EPW

    pltpu.sync_copy(den.at[0], dtab)
    pltpu.sync_copy(den.at[1], dtmp)

    def _d(i, _):
        sl = pl.ds(i * 16, 16)
        dtab[sl] = dtab[sl] + dtmp[sl]
        return 0
    lax.fori_loop(0, NPAD // 16, _d, 0)

    def _zrow(i, _):
        for k in range(D // 16):
            s0.rows_m[i, pl.ds(k * 16, 16)] = jnp.zeros((16,), jnp.float32)
        return 0
    lax.fori_loop(0, BE, _zrow, 0)
    for k in range(ROWS_PT // BE):
        pltpu.sync_copy(s0.rows_m, acc_sp.at[pl.ds(sid * ROWS_PT + k * BE, BE)])
    plsc.subcore_barrier()

    def idx_copies(j, S):
        base = base0 + j * BE
        return (
            pltpu.make_async_copy(senders.at[pl.ds(base, BE)], S.sidx, S.semi),
            pltpu.make_async_copy(receivers.at[pl.ds(base, BE)], S.ridx, S.semi),
            pltpu.make_async_copy(p_in.at[pl.ds(base, BE)], S.pbuf, S.semi),
        )

    def gather_copy(S):
        return pltpu.make_async_copy(m_t.at[S.sidx], S.rows_m, S.semg)

    def halfstep(j, sc, sn, next1, next2):
        if next1:
            for c in idx_copies(j + 1, sn):
                c.wait()
            gather_copy(sn).start()

        def _g(gi, _):
            sl = pl.ds(gi * 16, 16)
            dv = plsc.load_gather(dtab, [sc.ridx[sl]])
            sc.wbuf[sl] = sc.pbuf[sl] / dv
            return 0
        lax.fori_loop(0, BE // 16, _g, 0)

        gather_copy(sc).wait()

        def _row(i, _):
            wv = plsc.load_gather(sc.wbuf, [jnp.full((16,), i, jnp.int32)])
            for k in range(D // 16):
                sl = pl.ds(k * 16, 16)
                sc.rows_m[i, sl] = sc.rows_m[i, sl] * wv
            return 0
        lax.fori_loop(0, BE, _row, 0)

        pltpu.sync_copy(sc.rows_m, acc_sp.at[sc.ridx], add=True)
        if next2:
            for c in idx_copies(j + 2, sc):
                c.start()

    for c in idx_copies(0, s0):
        c.start()
    for c in idx_copies(0, s0):
        c.wait()
    gather_copy(s0).start()
    for c in idx_copies(1, s1):
        c.start()

    halfstep(0, s0, s1, True, True)

    def _pair(k, _):
        halfstep(2 * k + 1, s1, s0, True, True)
        halfstep(2 * k + 2, s0, s1, True, True)
        return 0
    lax.fori_loop(0, (NBLK - 3) // 2, _pair, 0)

    halfstep(NBLK - 2, s1, s0, True, False)
    halfstep(NBLK - 1, s0, s1, False, False)

    plsc.subcore_barrier()
    for k in range(ROWS_PT // BE):
        sl = pl.ds(sid * ROWS_PT + k * BE, BE)
        pltpu.sync_copy(acc_sp.at[sl], s0.rows_m)
        pltpu.sync_copy(s0.rows_m, acc_out.at[cid, sl])


@functools.cache
def _msg_phase():
    sets = []
    for _ in range(2):
        sets += [
            pltpu.VMEM((BE,), jnp.int32),
            pltpu.VMEM((BE,), jnp.int32),
            pltpu.VMEM((BE,), jnp.float32),
            pltpu.VMEM((BE,), jnp.float32),
            pltpu.VMEM((BE, D), jnp.float32),
        ]
    return pl.kernel(
        _msg_phase_body,
        out_type=jax.ShapeDtypeStruct((NC, NPAD, D), jnp.float32),
        mesh=_mesh(),
        scratch_types=sets + [
            pltpu.VMEM((NPAD,), jnp.float32),
            pltpu.VMEM((NPAD,), jnp.float32),
            pltpu.VMEM_SHARED((NPAD, D), jnp.float32),
        ] + [pltpu.SemaphoreType.DMA] * 4,
        compiler_params=pltpu.CompilerParams(needs_layout_passes=False),
    )


def kernel(nodes, edges, senders, receivers,
           W1, b1, W2, b2, W3, b3, W4, b4, W5, b5):
    s32 = senders.astype(jnp.int32)
    r32 = receivers.astype(jnp.int32)

    edges_t = edges.T
    a_t, b_t, m_t, a_sc, b_sc = _node_mm(nodes, W1, b1, W2, b2, W5, b5, W4)
    e_sc = _escal_mm(edges_t, W3, b3, W4, b4)

    g, p, den = _edge_phase()(
        s32, r32, a_t, b_t,
        a_sc.reshape(N), b_sc.reshape(N), e_sc.reshape(E))
    acc = _msg_phase()(s32, r32, m_t, p, den)
    ef = _ef_mm(g, edges_t, W3, b3)
    new_nodes = _sum_cores(acc)[:N]
    return new_nodes, ef


# speedup vs baseline: 15.5012x; 1.2089x over previous
import functools
from types import SimpleNamespace

import jax
import jax.numpy as jnp
from jax import lax
from jax.experimental import pallas as pl
from jax.experimental.pallas import tpu as pltpu
from jax.experimental.pallas import tpu_sc as plsc

N = 10000
E = 320000
D = 128
DE = 16

NC = 2
NS = 16
NW = NC * NS
EPW = E // NW
BE = 80
NBLK = EPW // BE
NPAD = 10240
ROWS_PT = NPAD // NS


@functools.cache
def _mesh():
    return plsc.VectorSubcoreMesh(
        core_axis_name="c", subcore_axis_name="s",
        num_cores=NC, num_subcores=NS)


def _node_mm_body(x_ref, w1, b1, w2, b2, w5, b5, w4,
                  a_out, b_out, m_out, as_out, bs_out):
    x = x_ref[...]
    a = jnp.dot(x, w1[...], preferred_element_type=jnp.float32) + b1[...]
    b = jnp.dot(x, w2[...], preferred_element_type=jnp.float32) + b2[...]
    m = jnp.dot(x, w5[...], preferred_element_type=jnp.float32) + b5[...]
    a_out[...] = a
    b_out[...] = b
    m_out[...] = m
    w4v = w4[...]
    blk = x.shape[0]
    as_out[...] = lax.dot_general(w4v, a, (((0,), (1,)), ((), ())),
                                  preferred_element_type=jnp.float32
                                  ).reshape(1, 1, blk)
    bs_out[...] = lax.dot_general(w4v, b, (((0,), (1,)), ((), ())),
                                  preferred_element_type=jnp.float32
                                  ).reshape(1, 1, blk)


def _escal_mm_body(et_ref, w3, b3, w4, b4, es_out):
    et = et_ref[...]
    blk = et.shape[1]
    w34 = jnp.dot(w3[...], w4[...], preferred_element_type=jnp.float32)
    c = jnp.dot(b3[...], w4[...], preferred_element_type=jnp.float32)
    es = jnp.sum(et * w34, axis=0)
    es_out[...] = (es + c[0, 0] + b4[0, 0]).reshape(1, 1, blk)


def _ef_mm_body(g_ref, et_ref, w3, b3, ef_out):
    et = et_ref[...]
    ef_out[...] = g_ref[...] + b3[...] + lax.dot_general(
        et, w3[...], (((0,), (0,)), ((), ())),
        preferred_element_type=jnp.float32)


def _sum_cores_body(acc_ref, out_ref):
    out_ref[...] = acc_ref[0] + acc_ref[1]


def _node_mm(nodes, W1, b1, W2, b2, W5, b5, W4):
    blk = 400
    grid = N // blk
    full = lambda shape: pl.BlockSpec(shape, lambda i: (0, 0))
    return pl.pallas_call(
        _node_mm_body,
        grid=(grid,),
        in_specs=[
            pl.BlockSpec((blk, D), lambda i: (i, 0)),
            full((D, D)), full((1, D)),
            full((D, D)), full((1, D)),
            full((D, D)), full((1, D)),
            full((D, 1)),
        ],
        out_specs=[
            pl.BlockSpec((blk, D), lambda i: (i, 0)),
            pl.BlockSpec((blk, D), lambda i: (i, 0)),
            pl.BlockSpec((blk, D), lambda i: (i, 0)),
            pl.BlockSpec((1, 1, blk), lambda i: (i, 0, 0)),
            pl.BlockSpec((1, 1, blk), lambda i: (i, 0, 0)),
        ],
        out_shape=[
            jax.ShapeDtypeStruct((N, D), jnp.float32),
            jax.ShapeDtypeStruct((N, D), jnp.float32),
            jax.ShapeDtypeStruct((N, D), jnp.float32),
            jax.ShapeDtypeStruct((grid, 1, blk), jnp.float32),
            jax.ShapeDtypeStruct((grid, 1, blk), jnp.float32),
        ],
    )(nodes, W1, b1.reshape(1, D), W2, b2.reshape(1, D),
      W5, b5.reshape(1, D), W4)


def _escal_mm(edges_t, W3, b3, W4, b4):
    blk = 12800
    grid = E // blk
    full = lambda shape: pl.BlockSpec(shape, lambda i: (0, 0))
    return pl.pallas_call(
        _escal_mm_body,
        grid=(grid,),
        in_specs=[
            pl.BlockSpec((DE, blk), lambda i: (0, i)),
            full((DE, D)), full((1, D)),
            full((D, 1)), full((1, 1)),
        ],
        out_specs=pl.BlockSpec((1, 1, blk), lambda i: (i, 0, 0)),
        out_shape=jax.ShapeDtypeStruct((grid, 1, blk), jnp.float32),
    )(edges_t, W3, b3.reshape(1, D), W4, b4.reshape(1, 1))


def _ef_mm(g, edges_t, W3, b3):
    blk = 2560
    grid = E // blk
    full = lambda shape: pl.BlockSpec(shape, lambda i: (0, 0))
    return pl.pallas_call(
        _ef_mm_body,
        grid=(grid,),
        in_specs=[
            pl.BlockSpec((blk, D), lambda i: (i, 0)),
            pl.BlockSpec((DE, blk), lambda i: (0, i)),
            full((DE, D)), full((1, D)),
        ],
        out_specs=pl.BlockSpec((blk, D), lambda i: (i, 0)),
        out_shape=jax.ShapeDtypeStruct((E, D), jnp.float32),
    )(g, edges_t, W3, b3.reshape(1, D))


def _sum_cores(acc):
    blk = 2048
    grid = NPAD // blk
    return pl.pallas_call(
        _sum_cores_body,
        grid=(grid,),
        in_specs=[pl.BlockSpec((NC, blk, D), lambda i: (0, i, 0))],
        out_specs=pl.BlockSpec((blk, D), lambda i: (i, 0)),
        out_shape=jax.ShapeDtypeStruct((NPAD, D), jnp.float32),
    )(acc)


def _edge_phase_body(senders, receivers, a_t, b_t, a_sc, b_sc, e_sc,
                     g_out, p_out, den_out, *scr):
    s0 = SimpleNamespace(sidx=scr[0], ridx=scr[1], escb=scr[2], pbuf=scr[3],
                         rsc=scr[4], rows_a=scr[5], rows_b=scr[6],
                         semi=scr[18], semg=scr[19], semw=scr[20],
                         semsc=scr[21])
    s1 = SimpleNamespace(sidx=scr[7], ridx=scr[8], escb=scr[9], pbuf=scr[10],
                         rsc=scr[11], rows_a=scr[12], rows_b=scr[13],
                         semi=scr[22], semg=scr[23], semw=scr[24],
                         semsc=scr[25])
    atab, btab, dtile, dsp = scr[14], scr[15], scr[16], scr[17]

    cid = lax.axis_index("c")
    sid = lax.axis_index("s")
    wid = sid * NC + cid
    base0 = wid * EPW

    def _z(i, _):
        dtile[pl.ds(i * 16, 16)] = jnp.zeros((16,), jnp.float32)
        return 0
    lax.fori_loop(0, ROWS_PT // 16, _z, 0)
    pltpu.sync_copy(dtile, dsp.at[pl.ds(sid * ROWS_PT, ROWS_PT)])

    pltpu.sync_copy(a_sc, atab.at[pl.ds(0, N)])
    pltpu.sync_copy(b_sc, btab.at[pl.ds(0, N)])
    plsc.subcore_barrier()

    def idx_copies(j, S):
        base = base0 + j * BE
        return (
            pltpu.make_async_copy(senders.at[pl.ds(base, BE)], S.sidx, S.semi),
            pltpu.make_async_copy(receivers.at[pl.ds(base, BE)], S.ridx, S.semi),
            pltpu.make_async_copy(e_sc.at[pl.ds(base, BE)], S.escb, S.semi),
        )

    def gather_copies(j, S):
        return (
            pltpu.make_async_copy(a_t.at[S.sidx], S.rows_a, S.semg),
            pltpu.make_async_copy(b_t.at[S.ridx], S.rows_b, S.semg),
        )

    def g_write(j, S):
        base = base0 + j * BE
        return pltpu.make_async_copy(S.rows_a, g_out.at[pl.ds(base, BE)], S.semw)

    def p_write(j, S):
        base = base0 + j * BE
        return pltpu.make_async_copy(S.pbuf, p_out.at[pl.ds(base, BE)], S.semw)

    def den_scatter(S):
        return pltpu.make_async_copy(S.pbuf, dsp.at[S.rsc], S.semsc)

    def drain_prev(j, S):
        g_write(j, S).wait()
        p_write(j, S).wait()
        den_scatter(S).wait()

    def halfstep(j, sc, sn, next1, next2, drainw):
        if next1:
            for c in idx_copies(j + 1, sn):
                c.wait()
            if drainw:
                drain_prev(j - 1, sn)
            for c in gather_copies(j + 1, sn):
                c.start()
        elif drainw:
            drain_prev(j - 1, sn)

        def _g(gi, _):
            sl = pl.ds(gi * 16, 16)
            av = plsc.load_gather(atab, [sc.sidx[sl]])
            bv = plsc.load_gather(btab, [sc.ridx[sl]])
            logit = av + bv + sc.escb[sl]
            logit = jnp.where(logit > 0, logit, logit * jnp.float32(0.01))
            sc.pbuf[sl] = jnp.exp(logit)
            return 0
        lax.fori_loop(0, BE // 16, _g, 0)

        for c in gather_copies(j, sc):
            c.wait()

        def _row(i, _):
            for k in range(D // 16):
                sl = pl.ds(k * 16, 16)
                sc.rows_a[i, sl] = sc.rows_a[i, sl] + sc.rows_b[i, sl]
            return 0
        lax.fori_loop(0, BE, _row, 0)

        g_write(j, sc).start()
        p_write(j, sc).start()
        for k in range(BE // 16):
            sl = pl.ds(k * 16, 16)
            sc.rsc[sl] = sc.ridx[sl]
        pltpu.async_copy(sc.pbuf, dsp.at[sc.rsc], sc.semsc, add=True)
        if next2:
            for c in idx_copies(j + 2, sc):
                c.start()

    for c in idx_copies(0, s0):
        c.start()
    for c in idx_copies(0, s0):
        c.wait()
    for c in gather_copies(0, s0):
        c.start()
    for c in idx_copies(1, s1):
        c.start()

    halfstep(0, s0, s1, True, True, False)

    def _pair(k, _):
        halfstep(2 * k + 1, s1, s0, True, True, True)
        halfstep(2 * k + 2, s0, s1, True, True, True)
        return 0
    lax.fori_loop(0, (NBLK - 3) // 2, _pair, 0)

    halfstep(NBLK - 2, s1, s0, True, False, True)
    halfstep(NBLK - 1, s0, s1, False, False, True)
    drain_prev(NBLK - 1, s0)

    plsc.subcore_barrier()
    pltpu.sync_copy(dsp.at[pl.ds(sid * ROWS_PT, ROWS_PT)], dtile)
    pltpu.sync_copy(dtile, den_out.at[cid, pl.ds(sid * ROWS_PT, ROWS_PT)])


@functools.cache
def _edge_phase():
    sets = []
    for _ in range(2):
        sets += [
            pltpu.VMEM((BE,), jnp.int32),
            pltpu.VMEM((BE,), jnp.int32),
            pltpu.VMEM((BE,), jnp.float32),
            pltpu.VMEM((BE,), jnp.float32),
            pltpu.VMEM((BE,), jnp.int32),
            pltpu.VMEM((BE, D), jnp.float32),
            pltpu.VMEM((BE, D), jnp.float32),
        ]
    return pl.kernel(
        _edge_phase_body,
        out_type=(
            jax.ShapeDtypeStruct((E, D), jnp.float32),
            jax.ShapeDtypeStruct((E,), jnp.float32),
            jax.ShapeDtypeStruct((NC, NPAD), jnp.float32),
        ),
        mesh=_mesh(),
        scratch_types=sets + [
            pltpu.VMEM((NPAD,), jnp.float32),
            pltpu.VMEM((NPAD,), jnp.float32),
            pltpu.VMEM((ROWS_PT,), jnp.float32),
            pltpu.VMEM_SHARED((NPAD,), jnp.float32),
        ] + [pltpu.SemaphoreType.DMA] * 8,
        compiler_params=pltpu.CompilerParams(needs_layout_passes=False),
    )


def _msg_phase_body(senders, receivers, m_t, p_in, den, acc_out, *scr):
    s0 = SimpleNamespace(sidx=scr[0], ridx=scr[1], pbuf=scr[2], wbuf=scr[3],
                         rsc=scr[4], rows_m=scr[5],
                         semi=scr[15], semg=scr[16], semsc=scr[17])
    s1 = SimpleNamespace(sidx=scr[6], ridx=scr[7], pbuf=scr[8], wbuf=scr[9],
                         rsc=scr[10], rows_m=scr[11],
                         semi=scr[18], semg=scr[19], semsc=scr[20])
    dtab, dtmp, acc_sp = scr[12], scr[13], scr[14]

    cid = lax.axis_index("c")
    sid = lax.axis_index("s")
    wid = sid * NC + cid
    base0 = wid * EPW

    pltpu.sync_copy(den.at[0], dtab)
    pltpu.sync_copy(den.at[1], dtmp)

    def _d(i, _):
        sl = pl.ds(i * 16, 16)
        dtab[sl] = dtab[sl] + dtmp[sl]
        return 0
    lax.fori_loop(0, NPAD // 16, _d, 0)

    def _zrow(i, _):
        for k in range(D // 16):
            s0.rows_m[i, pl.ds(k * 16, 16)] = jnp.zeros((16,), jnp.float32)
        return 0
    lax.fori_loop(0, BE, _zrow, 0)
    for k in range(ROWS_PT // BE):
        pltpu.sync_copy(s0.rows_m, acc_sp.at[pl.ds(sid * ROWS_PT + k * BE, BE)])
    plsc.subcore_barrier()

    def idx_copies(j, S):
        base = base0 + j * BE
        return (
            pltpu.make_async_copy(senders.at[pl.ds(base, BE)], S.sidx, S.semi),
            pltpu.make_async_copy(receivers.at[pl.ds(base, BE)], S.ridx, S.semi),
            pltpu.make_async_copy(p_in.at[pl.ds(base, BE)], S.pbuf, S.semi),
        )

    def gather_copy(S):
        return pltpu.make_async_copy(m_t.at[S.sidx], S.rows_m, S.semg)

    def acc_scatter(S):
        return pltpu.make_async_copy(S.rows_m, acc_sp.at[S.rsc], S.semsc)

    def halfstep(j, sc, sn, next1, next2, drainsc):
        if next1:
            for c in idx_copies(j + 1, sn):
                c.wait()
            if drainsc:
                acc_scatter(sn).wait()
            gather_copy(sn).start()
        elif drainsc:
            acc_scatter(sn).wait()

        def _g(gi, _):
            sl = pl.ds(gi * 16, 16)
            dv = plsc.load_gather(dtab, [sc.ridx[sl]])
            sc.wbuf[sl] = sc.pbuf[sl] / dv
            return 0
        lax.fori_loop(0, BE // 16, _g, 0)

        gather_copy(sc).wait()

        def _row(i, _):
            wv = plsc.load_gather(sc.wbuf, [jnp.full((16,), i, jnp.int32)])
            for k in range(D // 16):
                sl = pl.ds(k * 16, 16)
                sc.rows_m[i, sl] = sc.rows_m[i, sl] * wv
            return 0
        lax.fori_loop(0, BE, _row, 0)

        for k in range(BE // 16):
            sl = pl.ds(k * 16, 16)
            sc.rsc[sl] = sc.ridx[sl]
        pltpu.async_copy(sc.rows_m, acc_sp.at[sc.rsc], sc.semsc, add=True)
        if next2:
            for c in idx_copies(j + 2, sc):
                c.start()

    for c in idx_copies(0, s0):
        c.start()
    for c in idx_copies(0, s0):
        c.wait()
    gather_copy(s0).start()
    for c in idx_copies(1, s1):
        c.start()

    halfstep(0, s0, s1, True, True, False)

    def _pair(k, _):
        halfstep(2 * k + 1, s1, s0, True, True, True)
        halfstep(2 * k + 2, s0, s1, True, True, True)
        return 0
    lax.fori_loop(0, (NBLK - 3) // 2, _pair, 0)

    halfstep(NBLK - 2, s1, s0, True, False, True)
    halfstep(NBLK - 1, s0, s1, False, False, True)
    acc_scatter(s0).wait()

    plsc.subcore_barrier()
    for k in range(ROWS_PT // BE):
        sl = pl.ds(sid * ROWS_PT + k * BE, BE)
        pltpu.sync_copy(acc_sp.at[sl], s0.rows_m)
        pltpu.sync_copy(s0.rows_m, acc_out.at[cid, sl])


@functools.cache
def _msg_phase():
    sets = []
    for _ in range(2):
        sets += [
            pltpu.VMEM((BE,), jnp.int32),
            pltpu.VMEM((BE,), jnp.int32),
            pltpu.VMEM((BE,), jnp.float32),
            pltpu.VMEM((BE,), jnp.float32),
            pltpu.VMEM((BE,), jnp.int32),
            pltpu.VMEM((BE, D), jnp.float32),
        ]
    return pl.kernel(
        _msg_phase_body,
        out_type=jax.ShapeDtypeStruct((NC, NPAD, D), jnp.float32),
        mesh=_mesh(),
        scratch_types=sets + [
            pltpu.VMEM((NPAD,), jnp.float32),
            pltpu.VMEM((NPAD,), jnp.float32),
            pltpu.VMEM_SHARED((NPAD, D), jnp.float32),
        ] + [pltpu.SemaphoreType.DMA] * 6,
        compiler_params=pltpu.CompilerParams(needs_layout_passes=False),
    )


def kernel(nodes, edges, senders, receivers,
           W1, b1, W2, b2, W3, b3, W4, b4, W5, b5):
    s32 = senders.astype(jnp.int32)
    r32 = receivers.astype(jnp.int32)

    edges_t = edges.T
    a_t, b_t, m_t, a_sc, b_sc = _node_mm(nodes, W1, b1, W2, b2, W5, b5, W4)
    e_sc = _escal_mm(edges_t, W3, b3, W4, b4)

    g, p, den = _edge_phase()(
        s32, r32, a_t, b_t,
        a_sc.reshape(N), b_sc.reshape(N), e_sc.reshape(E))
    acc = _msg_phase()(s32, r32, m_t, p, den)
    ef = _ef_mm(g, edges_t, W3, b3)
    new_nodes = _sum_cores(acc)[:N]
    return new_nodes, ef


# speedup vs baseline: 16.1533x; 1.0421x over previous
import functools
from types import SimpleNamespace

import jax
import jax.numpy as jnp
from jax import lax
from jax.experimental import pallas as pl
from jax.experimental.pallas import tpu as pltpu
from jax.experimental.pallas import tpu_sc as plsc

N = 10000
E = 320000
D = 128
DE = 16

NC = 2
NS = 16
NW = NC * NS
EPW = E // NW
BE = 80
NBLK = EPW // BE
NPAD = 10240
ROWS_PT = NPAD // NS


@functools.cache
def _mesh():
    return plsc.VectorSubcoreMesh(
        core_axis_name="c", subcore_axis_name="s",
        num_cores=NC, num_subcores=NS)


def _node_mm_body(x_ref, et_ref, w1, b1, w2, b2, w5, b5, w4, w3, b3, b4,
                  a_out, b_out, m_out, as_out, bs_out, es_out):
    et = et_ref[...]
    w34 = jnp.dot(w3[...], w4[...], preferred_element_type=jnp.float32)
    c = jnp.dot(b3[...], w4[...], preferred_element_type=jnp.float32)
    es = jnp.sum(et * w34, axis=0)
    es_out[...] = (es + c[0, 0] + b4[0, 0]).reshape(1, 1, et.shape[1])
    x = x_ref[...]
    a = jnp.dot(x, w1[...], preferred_element_type=jnp.float32) + b1[...]
    b = jnp.dot(x, w2[...], preferred_element_type=jnp.float32) + b2[...]
    m = jnp.dot(x, w5[...], preferred_element_type=jnp.float32) + b5[...]
    a_out[...] = a
    b_out[...] = b
    m_out[...] = m
    w4v = w4[...]
    blk = x.shape[0]
    as_out[...] = lax.dot_general(w4v, a, (((0,), (1,)), ((), ())),
                                  preferred_element_type=jnp.float32
                                  ).reshape(1, 1, blk)
    bs_out[...] = lax.dot_general(w4v, b, (((0,), (1,)), ((), ())),
                                  preferred_element_type=jnp.float32
                                  ).reshape(1, 1, blk)


def _ef_mm_body(g_ref, et_ref, w3, b3, ef_out):
    et = et_ref[...]
    ef_out[...] = g_ref[...] + b3[...] + lax.dot_general(
        et, w3[...], (((0,), (0,)), ((), ())),
        preferred_element_type=jnp.float32)


def _sum_cores_body(acc_ref, out_ref):
    out_ref[...] = acc_ref[0] + acc_ref[1]


def _node_mm(nodes, edges_t, W1, b1, W2, b2, W5, b5, W4, W3, b3, b4):
    blk = 400
    eblk = E // (N // blk)
    grid = N // blk
    full = lambda shape: pl.BlockSpec(shape, lambda i: (0, 0))
    return pl.pallas_call(
        _node_mm_body,
        grid=(grid,),
        in_specs=[
            pl.BlockSpec((blk, D), lambda i: (i, 0)),
            pl.BlockSpec((DE, eblk), lambda i: (0, i)),
            full((D, D)), full((1, D)),
            full((D, D)), full((1, D)),
            full((D, D)), full((1, D)),
            full((D, 1)),
            full((DE, D)), full((1, D)), full((1, 1)),
        ],
        out_specs=[
            pl.BlockSpec((blk, D), lambda i: (i, 0)),
            pl.BlockSpec((blk, D), lambda i: (i, 0)),
            pl.BlockSpec((blk, D), lambda i: (i, 0)),
            pl.BlockSpec((1, 1, blk), lambda i: (i, 0, 0)),
            pl.BlockSpec((1, 1, blk), lambda i: (i, 0, 0)),
            pl.BlockSpec((1, 1, eblk), lambda i: (i, 0, 0)),
        ],
        out_shape=[
            jax.ShapeDtypeStruct((N, D), jnp.float32),
            jax.ShapeDtypeStruct((N, D), jnp.float32),
            jax.ShapeDtypeStruct((N, D), jnp.float32),
            jax.ShapeDtypeStruct((grid, 1, blk), jnp.float32),
            jax.ShapeDtypeStruct((grid, 1, blk), jnp.float32),
            jax.ShapeDtypeStruct((grid, 1, eblk), jnp.float32),
        ],
    )(nodes, edges_t, W1, b1.reshape(1, D), W2, b2.reshape(1, D),
      W5, b5.reshape(1, D), W4, W3, b3.reshape(1, D), b4.reshape(1, 1))


def _ef_mm(g, edges_t, W3, b3):
    blk = 2560
    grid = E // blk
    full = lambda shape: pl.BlockSpec(shape, lambda i: (0, 0))
    return pl.pallas_call(
        _ef_mm_body,
        grid=(grid,),
        in_specs=[
            pl.BlockSpec((blk, D), lambda i: (i, 0)),
            pl.BlockSpec((DE, blk), lambda i: (0, i)),
            full((DE, D)), full((1, D)),
        ],
        out_specs=pl.BlockSpec((blk, D), lambda i: (i, 0)),
        out_shape=jax.ShapeDtypeStruct((E, D), jnp.float32),
    )(g, edges_t, W3, b3.reshape(1, D))


def _sum_cores(acc):
    blk = 2048
    grid = NPAD // blk
    return pl.pallas_call(
        _sum_cores_body,
        grid=(grid,),
        in_specs=[pl.BlockSpec((NC, blk, D), lambda i: (0, i, 0))],
        out_specs=pl.BlockSpec((blk, D), lambda i: (i, 0)),
        out_shape=jax.ShapeDtypeStruct((NPAD, D), jnp.float32),
    )(acc)


def _edge_phase_body(senders, receivers, a_t, b_t, a_sc, b_sc, e_sc,
                     g_out, p_out, den_out, *scr):
    s0 = SimpleNamespace(sidx=scr[0], ridx=scr[1], escb=scr[2], pbuf=scr[3],
                         rsc=scr[4], rows_a=scr[5], rows_b=scr[6],
                         semi=scr[18], semg=scr[19], semw=scr[20],
                         semsc=scr[21])
    s1 = SimpleNamespace(sidx=scr[7], ridx=scr[8], escb=scr[9], pbuf=scr[10],
                         rsc=scr[11], rows_a=scr[12], rows_b=scr[13],
                         semi=scr[22], semg=scr[23], semw=scr[24],
                         semsc=scr[25])
    atab, btab, dtile, dsp = scr[14], scr[15], scr[16], scr[17]

    cid = lax.axis_index("c")
    sid = lax.axis_index("s")
    wid = sid * NC + cid
    base0 = wid * EPW

    def _z(i, _):
        dtile[pl.ds(i * 16, 16)] = jnp.zeros((16,), jnp.float32)
        return 0
    lax.fori_loop(0, ROWS_PT // 16, _z, 0)
    pltpu.sync_copy(dtile, dsp.at[pl.ds(sid * ROWS_PT, ROWS_PT)])

    pltpu.sync_copy(a_sc, atab.at[pl.ds(0, N)])
    pltpu.sync_copy(b_sc, btab.at[pl.ds(0, N)])
    plsc.subcore_barrier()

    def idx_copies(j, S):
        base = base0 + j * BE
        return (
            pltpu.make_async_copy(senders.at[pl.ds(base, BE)], S.sidx, S.semi),
            pltpu.make_async_copy(receivers.at[pl.ds(base, BE)], S.ridx, S.semi),
            pltpu.make_async_copy(e_sc.at[pl.ds(base, BE)], S.escb, S.semi),
        )

    def gather_copies(j, S):
        return (
            pltpu.make_async_copy(a_t.at[S.sidx], S.rows_a, S.semg),
            pltpu.make_async_copy(b_t.at[S.ridx], S.rows_b, S.semg),
        )

    def g_write(j, S):
        base = base0 + j * BE
        return pltpu.make_async_copy(S.rows_a, g_out.at[pl.ds(base, BE)], S.semw)

    def p_write(j, S):
        base = base0 + j * BE
        return pltpu.make_async_copy(S.pbuf, p_out.at[pl.ds(base, BE)], S.semw)

    def den_scatter(S):
        return pltpu.make_async_copy(S.pbuf, dsp.at[S.rsc], S.semsc)

    def drain_prev(j, S):
        g_write(j, S).wait()
        p_write(j, S).wait()
        den_scatter(S).wait()

    def halfstep(j, sc, sn, next1, next2, drainw):
        if next1:
            for c in idx_copies(j + 1, sn):
                c.wait()
            if drainw:
                drain_prev(j - 1, sn)
            for c in gather_copies(j + 1, sn):
                c.start()
        elif drainw:
            drain_prev(j - 1, sn)

        def _g(gi, _):
            sl = pl.ds(gi * 16, 16)
            av = plsc.load_gather(atab, [sc.sidx[sl]])
            bv = plsc.load_gather(btab, [sc.ridx[sl]])
            logit = av + bv + sc.escb[sl]
            logit = jnp.where(logit > 0, logit, logit * jnp.float32(0.01))
            sc.pbuf[sl] = jnp.exp(logit)
            return 0
        lax.fori_loop(0, BE // 16, _g, 0)

        for c in gather_copies(j, sc):
            c.wait()

        def _row(i, _):
            for u in range(4):
                r = i * 4 + u
                for k in range(D // 16):
                    sl = pl.ds(k * 16, 16)
                    sc.rows_a[r, sl] = sc.rows_a[r, sl] + sc.rows_b[r, sl]
            return 0
        lax.fori_loop(0, BE // 4, _row, 0)

        g_write(j, sc).start()
        p_write(j, sc).start()
        for k in range(BE // 16):
            sl = pl.ds(k * 16, 16)
            sc.rsc[sl] = sc.ridx[sl]
        pltpu.async_copy(sc.pbuf, dsp.at[sc.rsc], sc.semsc, add=True)
        if next2:
            for c in idx_copies(j + 2, sc):
                c.start()

    for c in idx_copies(0, s0):
        c.start()
    for c in idx_copies(0, s0):
        c.wait()
    for c in gather_copies(0, s0):
        c.start()
    for c in idx_copies(1, s1):
        c.start()

    halfstep(0, s0, s1, True, True, False)

    def _pair(k, _):
        halfstep(2 * k + 1, s1, s0, True, True, True)
        halfstep(2 * k + 2, s0, s1, True, True, True)
        return 0
    lax.fori_loop(0, (NBLK - 3) // 2, _pair, 0)

    halfstep(NBLK - 2, s1, s0, True, False, True)
    halfstep(NBLK - 1, s0, s1, False, False, True)
    drain_prev(NBLK - 1, s0)

    plsc.subcore_barrier()
    pltpu.sync_copy(dsp.at[pl.ds(sid * ROWS_PT, ROWS_PT)], dtile)
    pltpu.sync_copy(dtile, den_out.at[cid, pl.ds(sid * ROWS_PT, ROWS_PT)])


@functools.cache
def _edge_phase():
    sets = []
    for _ in range(2):
        sets += [
            pltpu.VMEM((BE,), jnp.int32),
            pltpu.VMEM((BE,), jnp.int32),
            pltpu.VMEM((BE,), jnp.float32),
            pltpu.VMEM((BE,), jnp.float32),
            pltpu.VMEM((BE,), jnp.int32),
            pltpu.VMEM((BE, D), jnp.float32),
            pltpu.VMEM((BE, D), jnp.float32),
        ]
    return pl.kernel(
        _edge_phase_body,
        out_type=(
            jax.ShapeDtypeStruct((E, D), jnp.float32),
            jax.ShapeDtypeStruct((E,), jnp.float32),
            jax.ShapeDtypeStruct((NC, NPAD), jnp.float32),
        ),
        mesh=_mesh(),
        scratch_types=sets + [
            pltpu.VMEM((NPAD,), jnp.float32),
            pltpu.VMEM((NPAD,), jnp.float32),
            pltpu.VMEM((ROWS_PT,), jnp.float32),
            pltpu.VMEM_SHARED((NPAD,), jnp.float32),
        ] + [pltpu.SemaphoreType.DMA] * 8,
        compiler_params=pltpu.CompilerParams(needs_layout_passes=False),
    )


def _msg_phase_body(senders, receivers, m_t, p_in, den, acc_out, *scr):
    s0 = SimpleNamespace(sidx=scr[0], ridx=scr[1], pbuf=scr[2], wbuf=scr[3],
                         rsc=scr[4], rows_m=scr[5],
                         semi=scr[15], semg=scr[16], semsc=scr[17])
    s1 = SimpleNamespace(sidx=scr[6], ridx=scr[7], pbuf=scr[8], wbuf=scr[9],
                         rsc=scr[10], rows_m=scr[11],
                         semi=scr[18], semg=scr[19], semsc=scr[20])
    dtab, dtmp, acc_sp = scr[12], scr[13], scr[14]

    cid = lax.axis_index("c")
    sid = lax.axis_index("s")
    wid = sid * NC + cid
    base0 = wid * EPW

    pltpu.sync_copy(den.at[0], dtab)
    pltpu.sync_copy(den.at[1], dtmp)

    def _d(i, _):
        sl = pl.ds(i * 16, 16)
        dtab[sl] = dtab[sl] + dtmp[sl]
        return 0
    lax.fori_loop(0, NPAD // 16, _d, 0)

    def _zrow(i, _):
        for k in range(D // 16):
            s0.rows_m[i, pl.ds(k * 16, 16)] = jnp.zeros((16,), jnp.float32)
        return 0
    lax.fori_loop(0, BE, _zrow, 0)
    for k in range(ROWS_PT // BE):
        pltpu.sync_copy(s0.rows_m, acc_sp.at[pl.ds(sid * ROWS_PT + k * BE, BE)])
    plsc.subcore_barrier()

    def idx_copies(j, S):
        base = base0 + j * BE
        return (
            pltpu.make_async_copy(senders.at[pl.ds(base, BE)], S.sidx, S.semi),
            pltpu.make_async_copy(receivers.at[pl.ds(base, BE)], S.ridx, S.semi),
            pltpu.make_async_copy(p_in.at[pl.ds(base, BE)], S.pbuf, S.semi),
        )

    def gather_copy(S):
        return pltpu.make_async_copy(m_t.at[S.sidx], S.rows_m, S.semg)

    def acc_scatter(S):
        return pltpu.make_async_copy(S.rows_m, acc_sp.at[S.rsc], S.semsc)

    def halfstep(j, sc, sn, next1, next2, drainsc):
        if next1:
            for c in idx_copies(j + 1, sn):
                c.wait()
            if drainsc:
                acc_scatter(sn).wait()
            gather_copy(sn).start()
        elif drainsc:
            acc_scatter(sn).wait()

        def _g(gi, _):
            sl = pl.ds(gi * 16, 16)
            dv = plsc.load_gather(dtab, [sc.ridx[sl]])
            sc.wbuf[sl] = sc.pbuf[sl] / dv
            return 0
        lax.fori_loop(0, BE // 16, _g, 0)

        gather_copy(sc).wait()

        def _row(i, _):
            for u in range(4):
                r = i * 4 + u
                wv = plsc.load_gather(sc.wbuf, [jnp.full((16,), r, jnp.int32)])
                for k in range(D // 16):
                    sl = pl.ds(k * 16, 16)
                    sc.rows_m[r, sl] = sc.rows_m[r, sl] * wv
            return 0
        lax.fori_loop(0, BE // 4, _row, 0)

        for k in range(BE // 16):
            sl = pl.ds(k * 16, 16)
            sc.rsc[sl] = sc.ridx[sl]
        pltpu.async_copy(sc.rows_m, acc_sp.at[sc.rsc], sc.semsc, add=True)
        if next2:
            for c in idx_copies(j + 2, sc):
                c.start()

    for c in idx_copies(0, s0):
        c.start()
    for c in idx_copies(0, s0):
        c.wait()
    gather_copy(s0).start()
    for c in idx_copies(1, s1):
        c.start()

    halfstep(0, s0, s1, True, True, False)

    def _pair(k, _):
        halfstep(2 * k + 1, s1, s0, True, True, True)
        halfstep(2 * k + 2, s0, s1, True, True, True)
        return 0
    lax.fori_loop(0, (NBLK - 3) // 2, _pair, 0)

    halfstep(NBLK - 2, s1, s0, True, False, True)
    halfstep(NBLK - 1, s0, s1, False, False, True)
    acc_scatter(s0).wait()

    plsc.subcore_barrier()
    for k in range(ROWS_PT // BE):
        sl = pl.ds(sid * ROWS_PT + k * BE, BE)
        pltpu.sync_copy(acc_sp.at[sl], s0.rows_m)
        pltpu.sync_copy(s0.rows_m, acc_out.at[cid, sl])


@functools.cache
def _msg_phase():
    sets = []
    for _ in range(2):
        sets += [
            pltpu.VMEM((BE,), jnp.int32),
            pltpu.VMEM((BE,), jnp.int32),
            pltpu.VMEM((BE,), jnp.float32),
            pltpu.VMEM((BE,), jnp.float32),
            pltpu.VMEM((BE,), jnp.int32),
            pltpu.VMEM((BE, D), jnp.float32),
        ]
    return pl.kernel(
        _msg_phase_body,
        out_type=jax.ShapeDtypeStruct((NC, NPAD, D), jnp.float32),
        mesh=_mesh(),
        scratch_types=sets + [
            pltpu.VMEM((NPAD,), jnp.float32),
            pltpu.VMEM((NPAD,), jnp.float32),
            pltpu.VMEM_SHARED((NPAD, D), jnp.float32),
        ] + [pltpu.SemaphoreType.DMA] * 6,
        compiler_params=pltpu.CompilerParams(needs_layout_passes=False),
    )


def kernel(nodes, edges, senders, receivers,
           W1, b1, W2, b2, W3, b3, W4, b4, W5, b5):
    s32 = senders.astype(jnp.int32)
    r32 = receivers.astype(jnp.int32)

    edges_t = edges.T
    a_t, b_t, m_t, a_sc, b_sc, e_sc = _node_mm(
        nodes, edges_t, W1, b1, W2, b2, W5, b5, W4, W3, b3, b4)

    g, p, den = _edge_phase()(
        s32, r32, a_t, b_t,
        a_sc.reshape(N), b_sc.reshape(N), e_sc.reshape(E))
    acc = _msg_phase()(s32, r32, m_t, p, den)
    ef = _ef_mm(g, edges_t, W3, b3)
    new_nodes = _sum_cores(acc)[:N]
    return new_nodes, ef
